# Initial kernel scaffold; baseline (speedup 1.0000x reference)
import functools

import jax
import jax.numpy as jnp
from jax import lax
from jax.experimental import pallas as pl
from jax.experimental.pallas import tpu as pltpu
from jax.experimental.pallas import tpu_sc as plsc

N = 10000
NP = 10240
DIN = 128
DH = 100
DP = 112
E = 320000
NC, NS, LB = 2, 16, 128
NBLK = -(-E // (NC * NS * LB))
EP = NC * NS * NBLK * LB
PAD = N
RB = NP // NS
NZ = RB // LB

f32 = jnp.float32
i32 = jnp.int32
BLK = 1280


def _mm_body(x_ref, w_ref, o_ref):
    o_ref[...] = jnp.dot(x_ref[...], w_ref[...], preferred_element_type=f32)


def _tc_matmul(xp, w):
    return pl.pallas_call(
        _mm_body,
        grid=(NP // BLK,),
        in_specs=[pl.BlockSpec((BLK, DIN), lambda i: (i, 0)),
                  pl.BlockSpec((DIN, DP), lambda i: (0, 0))],
        out_specs=pl.BlockSpec((BLK, DP), lambda i: (i, 0)),
        out_shape=jax.ShapeDtypeStruct((NP, DP), f32),
    )(xp, w)


def _prep_body(degp_ref, h_ref, hp_ref, dinv_ref):
    deg = degp_ref[0, :, 0:1] + degp_ref[1, :, 0:1] + 1.0
    dinv = 1.0 / jnp.sqrt(deg)
    dinv_ref[...] = dinv
    hp_ref[...] = h_ref[...] * dinv


def _tc_prep(degp, h):
    return pl.pallas_call(
        _prep_body,
        grid=(NP // BLK,),
        in_specs=[pl.BlockSpec((NC, BLK, 16), lambda i: (0, i, 0)),
                  pl.BlockSpec((BLK, DP), lambda i: (i, 0))],
        out_specs=[pl.BlockSpec((BLK, DP), lambda i: (i, 0)),
                   pl.BlockSpec((BLK, 1), lambda i: (i, 0))],
        out_shape=[jax.ShapeDtypeStruct((NP, DP), f32),
                   jax.ShapeDtypeStruct((NP, 1), f32)],
    )(degp, h)


def _mid_body(msg_ref, h_ref, dinv_ref, b1_ref, w2_ref, zp16_ref, z_ref):
    dinv = dinv_ref[...]
    ssum = msg_ref[0] + msg_ref[1]
    h1 = jnp.maximum(dinv * ssum + (dinv * dinv) * h_ref[...] + b1_ref[...],
                     0.0)
    z = jnp.dot(h1, w2_ref[...], preferred_element_type=f32)
    z_ref[...] = z
    zp = dinv * z
    col = lax.broadcasted_iota(i32, zp16_ref.shape, 1)
    zp16_ref[...] = jnp.where(col == 0, zp, 0.0)


def _tc_mid(msg1, h, dinv, b1p, w2p):
    return pl.pallas_call(
        _mid_body,
        grid=(NP // BLK,),
        in_specs=[pl.BlockSpec((NC, BLK, DP), lambda i: (0, i, 0)),
                  pl.BlockSpec((BLK, DP), lambda i: (i, 0)),
                  pl.BlockSpec((BLK, 1), lambda i: (i, 0)),
                  pl.BlockSpec((1, DP), lambda i: (0, 0)),
                  pl.BlockSpec((DP, 1), lambda i: (0, 0))],
        out_specs=[pl.BlockSpec((BLK, 16), lambda i: (i, 0)),
                   pl.BlockSpec((BLK, 1), lambda i: (i, 0))],
        out_shape=[jax.ShapeDtypeStruct((NP, 16), f32),
                   jax.ShapeDtypeStruct((NP, 1), f32)],
    )(msg1, h, dinv, b1p, w2p)


def _fin_body(msg_ref, z_ref, dinv_ref, b2_ref, o_ref):
    dinv = dinv_ref[...]
    s = msg_ref[0, :, 0:1] + msg_ref[1, :, 0:1]
    o_ref[...] = dinv * s + (dinv * dinv) * z_ref[...] + b2_ref[...]


def _tc_fin(msg2, z, dinv, b2r):
    return pl.pallas_call(
        _fin_body,
        grid=(NP // BLK,),
        in_specs=[pl.BlockSpec((NC, BLK, 16), lambda i: (0, i, 0)),
                  pl.BlockSpec((BLK, 1), lambda i: (i, 0)),
                  pl.BlockSpec((BLK, 1), lambda i: (i, 0)),
                  pl.BlockSpec((1, 1), lambda i: (0, 0))],
        out_specs=pl.BlockSpec((BLK, 1), lambda i: (i, 0)),
        out_shape=jax.ShapeDtypeStruct((NP, 1), f32),
    )(msg2, z, dinv, b2r)


def _sc_mesh():
    return plsc.VectorSubcoreMesh(core_axis_name="c", subcore_axis_name="s")


def _sc_hist(dsti):
    @functools.partial(
        pl.kernel,
        out_type=jax.ShapeDtypeStruct((NC, NP, 16), f32),
        mesh=_sc_mesh(),
        scratch_types=[
            pltpu.VMEM((LB, 16), f32),
            pltpu.VMEM((LB, 16), f32),
            pltpu.VMEM((LB,), i32),
            pltpu.VMEM_SHARED((NP, 16), f32),
        ],
    )
    def k(dsti_hbm, out_hbm, zbuf, obuf, didx, acc):
        c = lax.axis_index("c")
        s = lax.axis_index("s")
        zero = jnp.zeros((16,), f32)
        onerow = jnp.where(lax.iota(i32, 16) == 0,
                           jnp.float32(1.0), jnp.float32(0.0))

        @pl.loop(0, LB)
        def _(r):
            zbuf[r] = zero
            obuf[r] = onerow

        for i in range(NZ):
            pltpu.sync_copy(zbuf, acc.at[pl.ds(s * RB + i * LB, LB)])
        plsc.subcore_barrier()

        @pl.loop(0, NBLK)
        def _(j):
            pltpu.sync_copy(dsti_hbm.at[c, s, j], didx)
            pltpu.sync_copy(obuf, acc.at[didx], add=True)

        plsc.subcore_barrier()
        for i in range(NZ):
            sl = pl.ds(s * RB + i * LB, LB)
            pltpu.sync_copy(acc.at[sl], out_hbm.at[c, sl])

    return k(dsti)


def _make_sc_msg(dp):
    @functools.partial(
        pl.kernel,
        out_type=jax.ShapeDtypeStruct((NC, NP, dp), f32),
        mesh=_sc_mesh(),
        scratch_types=[
            pltpu.VMEM((LB, dp), f32),
            pltpu.VMEM((LB,), i32),
            pltpu.VMEM((LB,), i32),
            pltpu.VMEM_SHARED((NP, dp), f32),
            pltpu.SemaphoreType.DMA,
        ],
    )
    def k(v_hbm, sidx_hbm, didx_hbm, out_hbm, rows, sidx, didx, acc, sem):
        c = lax.axis_index("c")
        s = lax.axis_index("s")
        zero = jnp.zeros((16,), f32)

        @pl.loop(0, LB)
        def _(r):
            @pl.loop(0, dp, step=16)
            def _(cc):
                rows[r, pl.ds(cc, 16)] = zero

        for i in range(NZ):
            pltpu.sync_copy(rows, acc.at[pl.ds(s * RB + i * LB, LB)])
        plsc.subcore_barrier()

        @pl.loop(0, NBLK)
        def _(j):
            pltpu.sync_copy(sidx_hbm.at[c, s, j], sidx)
            pltpu.sync_copy(didx_hbm.at[c, s, j], didx)
            pltpu.async_copy(v_hbm.at[sidx], rows, sem).wait()
            pltpu.sync_copy(rows, acc.at[didx], add=True)

        plsc.subcore_barrier()
        for i in range(NZ):
            sl = pl.ds(s * RB + i * LB, LB)
            pltpu.sync_copy(acc.at[sl], out_hbm.at[c, sl])

    return k


_sc_msg_wide = _make_sc_msg(DP)
_sc_msg_thin = _make_sc_msg(16)


def kernel(x, edge_index, W1, b1, W2, b2):
    ei = edge_index.astype(i32)
    pad = jnp.full((EP - E,), PAD, i32)
    src = jnp.concatenate([ei[0], pad]).reshape(NC, NS, NBLK, LB)
    dst = jnp.concatenate([ei[1], pad]).reshape(NC, NS, NBLK, LB)

    xp = jnp.pad(x, ((0, NP - N), (0, 0)))
    W1p = jnp.pad(W1, ((0, 0), (0, DP - DH)))
    b1p = jnp.pad(b1, (0, DP - DH)).reshape(1, DP)
    W2p = jnp.pad(W2, ((0, DP - DH), (0, 0)))
    b2r = b2.reshape(1, 1)

    degp = _sc_hist(dst)
    h = _tc_matmul(xp, W1p)
    hp, dinv = _tc_prep(degp, h)
    msg1 = _sc_msg_wide(hp, src, dst)
    zp16, z = _tc_mid(msg1, h, dinv, b1p, W2p)
    msg2 = _sc_msg_thin(zp16, src, dst)
    out = _tc_fin(msg2, z, dinv, b2r)
    return out[:N]


# speedup vs baseline: 11.9860x; 11.9860x over previous
import functools

import jax
import jax.numpy as jnp
from jax import lax
from jax.experimental import pallas as pl
from jax.experimental.pallas import tpu as pltpu
from jax.experimental.pallas import tpu_sc as plsc

N = 10000
NP = 10240
DIN = 128
DH = 100
DP = 128
E = 320000
NC, NS, LB = 2, 16, 128
NW = NC * NS
NBLK = -(-E // (NW * LB))
EP = NW * NBLK * LB
PAD = N
RB = NP // NS
NZ = RB // LB

f32 = jnp.float32
i32 = jnp.int32
BLK = 1280


def _mm_body(x_ref, w_ref, o_ref):
    o_ref[...] = jnp.dot(x_ref[...], w_ref[...], preferred_element_type=f32)


def _tc_matmul(xp, w):
    return pl.pallas_call(
        _mm_body,
        grid=(NP // BLK,),
        in_specs=[pl.BlockSpec((BLK, DIN), lambda i: (i, 0)),
                  pl.BlockSpec((DIN, DP), lambda i: (0, 0))],
        out_specs=pl.BlockSpec((BLK, DP), lambda i: (i, 0)),
        out_shape=jax.ShapeDtypeStruct((NP, DP), f32),
    )(xp, w)


def _colsum(part):
    ones = jnp.ones((NW, 1), f32)
    return lax.dot_general(part, ones, (((0,), (0,)), ((), ())),
                           preferred_element_type=f32)


def _prep_body(degp_ref, h_ref, hp_ref, dinv_ref):
    deg = _colsum(degp_ref[...]) + 1.0
    dinv = 1.0 / jnp.sqrt(deg)
    dinv_ref[...] = dinv
    hp_ref[...] = h_ref[...] * dinv


def _tc_prep(degp, h):
    return pl.pallas_call(
        _prep_body,
        grid=(NP // BLK,),
        in_specs=[pl.BlockSpec((NW, BLK), lambda i: (0, i)),
                  pl.BlockSpec((BLK, DP), lambda i: (i, 0))],
        out_specs=[pl.BlockSpec((BLK, DP), lambda i: (i, 0)),
                   pl.BlockSpec((BLK, 1), lambda i: (i, 0))],
        out_shape=[jax.ShapeDtypeStruct((NP, DP), f32),
                   jax.ShapeDtypeStruct((NP, 1), f32)],
    )(degp, h)


def _mid_body(msg_ref, h_ref, dinv_ref, b1_ref, w2_ref, zp_ref, z_ref):
    dinv = dinv_ref[...]
    ssum = msg_ref[0] + msg_ref[1]
    h1 = jnp.maximum(dinv * ssum + (dinv * dinv) * h_ref[...] + b1_ref[...],
                     0.0)
    z = jnp.dot(h1, w2_ref[...], preferred_element_type=f32)
    z_ref[...] = z
    zp_ref[...] = dinv * z


def _tc_mid(msg1, h, dinv, b1p, w2p):
    return pl.pallas_call(
        _mid_body,
        grid=(NP // BLK,),
        in_specs=[pl.BlockSpec((NC, BLK, DP), lambda i: (0, i, 0)),
                  pl.BlockSpec((BLK, DP), lambda i: (i, 0)),
                  pl.BlockSpec((BLK, 1), lambda i: (i, 0)),
                  pl.BlockSpec((1, DP), lambda i: (0, 0)),
                  pl.BlockSpec((DP, 1), lambda i: (0, 0))],
        out_specs=[pl.BlockSpec((BLK, 1), lambda i: (i, 0)),
                   pl.BlockSpec((BLK, 1), lambda i: (i, 0))],
        out_shape=[jax.ShapeDtypeStruct((NP, 1), f32),
                   jax.ShapeDtypeStruct((NP, 1), f32)],
    )(msg1, h, dinv, b1p, w2p)


def _fin_body(msg_ref, z_ref, dinv_ref, b2_ref, o_ref):
    dinv = dinv_ref[...]
    s = _colsum(msg_ref[...])
    o_ref[...] = dinv * s + (dinv * dinv) * z_ref[...] + b2_ref[...]


def _tc_fin(msg2, z, dinv, b2r):
    return pl.pallas_call(
        _fin_body,
        grid=(NP // BLK,),
        in_specs=[pl.BlockSpec((NW, BLK), lambda i: (0, i)),
                  pl.BlockSpec((BLK, 1), lambda i: (i, 0)),
                  pl.BlockSpec((BLK, 1), lambda i: (i, 0)),
                  pl.BlockSpec((1, 1), lambda i: (0, 0))],
        out_specs=pl.BlockSpec((BLK, 1), lambda i: (i, 0)),
        out_shape=jax.ShapeDtypeStruct((NP, 1), f32),
    )(msg2, z, dinv, b2r)


def _sc_mesh():
    return plsc.VectorSubcoreMesh(core_axis_name="c", subcore_axis_name="s")


@functools.cache
def _make_sc_scalar_msg():
    @functools.partial(
        pl.kernel,
        out_type=jax.ShapeDtypeStruct((NW, NP), f32),
        mesh=_sc_mesh(),
        compiler_params=pltpu.CompilerParams(needs_layout_passes=False),
        scratch_types=[
            pltpu.VMEM((NP,), f32),
            pltpu.VMEM((NP,), f32),
            pltpu.VMEM((LB,), i32),
            pltpu.VMEM((LB,), i32),
        ],
    )
    def k(vals_hbm, sidx_hbm, didx_hbm, out_hbm, vals, acc, sidx, didx):
        c = lax.axis_index("c")
        s = lax.axis_index("s")
        wid = c * NS + s
        zero = jnp.zeros((16,), f32)

        pltpu.sync_copy(vals_hbm, vals)

        @pl.loop(0, NP, step=16)
        def _(i):
            acc[pl.ds(i, 16)] = zero

        @pl.loop(0, NBLK)
        def _(j):
            pltpu.sync_copy(sidx_hbm.at[c, s, j], sidx)
            pltpu.sync_copy(didx_hbm.at[c, s, j], didx)

            @pl.loop(0, LB, step=16)
            def _(kk):
                sv = sidx[pl.ds(kk, 16)]
                dv = didx[pl.ds(kk, 16)]
                v = plsc.load_gather(vals, [sv])

                def cond(rem):
                    return jnp.any(rem)

                def body(rem):
                    _, last = plsc.scan_count(dv, mask=rem)
                    plsc.addupdate_scatter(acc, [dv], v, mask=last)
                    return rem & ~last

                lax.while_loop(cond, body,
                               jnp.full((16,), True, jnp.bool_))

        pltpu.sync_copy(acc, out_hbm.at[wid])

    return k


@functools.cache
def _make_sc_row_msg():
    @functools.partial(
        pl.kernel,
        out_type=jax.ShapeDtypeStruct((NC, NP, DP), f32),
        mesh=_sc_mesh(),
        scratch_types=[
            pltpu.VMEM((LB, DP), f32),
            pltpu.VMEM((LB,), i32),
            pltpu.VMEM((LB,), i32),
            pltpu.VMEM_SHARED((NP, DP), f32),
            pltpu.SemaphoreType.DMA,
        ],
    )
    def k(v_hbm, sidx_hbm, didx_hbm, out_hbm, rows, sidx, didx, acc, sem):
        c = lax.axis_index("c")
        s = lax.axis_index("s")
        zero = jnp.zeros((16,), f32)

        @pl.loop(0, LB)
        def _(r):
            @pl.loop(0, DP, step=16)
            def _(cc):
                rows[r, pl.ds(cc, 16)] = zero

        for i in range(NZ):
            pltpu.sync_copy(rows, acc.at[pl.ds(s * RB + i * LB, LB)])
        plsc.subcore_barrier()

        @pl.loop(0, NBLK)
        def _(j):
            pltpu.sync_copy(sidx_hbm.at[c, s, j], sidx)
            pltpu.sync_copy(didx_hbm.at[c, s, j], didx)
            pltpu.async_copy(v_hbm.at[sidx], rows, sem).wait()
            pltpu.sync_copy(rows, acc.at[didx], add=True)

        plsc.subcore_barrier()
        for i in range(NZ):
            sl = pl.ds(s * RB + i * LB, LB)
            pltpu.sync_copy(acc.at[sl], out_hbm.at[c, sl])

    return k


def kernel(x, edge_index, W1, b1, W2, b2):
    ei = edge_index.astype(i32)
    pad = jnp.full((EP - E,), PAD, i32)
    src = jnp.concatenate([ei[0], pad]).reshape(NC, NS, NBLK, LB)
    dst = jnp.concatenate([ei[1], pad]).reshape(NC, NS, NBLK, LB)

    xp = jnp.pad(x, ((0, NP - N), (0, 0)))
    W1p = jnp.pad(W1, ((0, 0), (0, DP - DH)))
    b1p = jnp.pad(b1, (0, DP - DH)).reshape(1, DP)
    W2p = jnp.pad(W2, ((0, DP - DH), (0, 0)))
    b2r = b2.reshape(1, 1)
    ones_n = jnp.ones((NP,), f32)

    degp = _make_sc_scalar_msg()(ones_n, dst, dst)
    h = _tc_matmul(xp, W1p)
    hp, dinv = _tc_prep(degp, h)
    msg1 = _make_sc_row_msg()(hp, src, dst)
    zp, z = _tc_mid(msg1, h, dinv, b1p, W2p)
    msg2 = _make_sc_scalar_msg()(zp.reshape(NP), src, dst)
    out = _tc_fin(msg2, z, dinv, b2r)
    return out[:N]


# speedup vs baseline: 16.8658x; 1.4071x over previous
import functools

import jax
import jax.numpy as jnp
from jax import lax
from jax.experimental import pallas as pl
from jax.experimental.pallas import tpu as pltpu
from jax.experimental.pallas import tpu_sc as plsc

N = 10000
NP = 10240
DIN = 128
DH = 100
DP = 128
E = 320000
NC, NS, LB = 2, 16, 128
NW = NC * NS
NBLK = -(-E // (NW * LB))
EP = NW * NBLK * LB
PAD = N
RB = NP // NS
NZ = RB // LB

f32 = jnp.float32
i32 = jnp.int32
BLK = 1280


def _mm_body(x_ref, w_ref, o_ref):
    o_ref[...] = jnp.dot(x_ref[...], w_ref[...], preferred_element_type=f32)


def _tc_matmul(xp, w):
    return pl.pallas_call(
        _mm_body,
        grid=(NP // BLK,),
        in_specs=[pl.BlockSpec((BLK, DIN), lambda i: (i, 0)),
                  pl.BlockSpec((DIN, DP), lambda i: (0, 0))],
        out_specs=pl.BlockSpec((BLK, DP), lambda i: (i, 0)),
        out_shape=jax.ShapeDtypeStruct((NP, DP), f32),
    )(xp, w)


def _colsum(part):
    ones = jnp.ones((NW, 1), f32)
    return lax.dot_general(part, ones, (((0,), (0,)), ((), ())),
                           preferred_element_type=f32)


def _prep_body(degp_ref, h_ref, hp_ref, dinv_ref):
    deg = _colsum(degp_ref[...]) + 1.0
    dinv = 1.0 / jnp.sqrt(deg)
    dinv_ref[...] = dinv
    hp_ref[...] = h_ref[...] * dinv


def _tc_prep(degp, h):
    return pl.pallas_call(
        _prep_body,
        grid=(NP // BLK,),
        in_specs=[pl.BlockSpec((NW, BLK), lambda i: (0, i)),
                  pl.BlockSpec((BLK, DP), lambda i: (i, 0))],
        out_specs=[pl.BlockSpec((BLK, DP), lambda i: (i, 0)),
                   pl.BlockSpec((BLK, 1), lambda i: (i, 0))],
        out_shape=[jax.ShapeDtypeStruct((NP, DP), f32),
                   jax.ShapeDtypeStruct((NP, 1), f32)],
    )(degp, h)


def _mid_body(msg_ref, h_ref, dinv_ref, b1_ref, w2_ref, zp_ref, z_ref):
    dinv = dinv_ref[...]
    ssum = msg_ref[0] + msg_ref[1]
    h1 = jnp.maximum(dinv * ssum + (dinv * dinv) * h_ref[...] + b1_ref[...],
                     0.0)
    z = jnp.dot(h1, w2_ref[...], preferred_element_type=f32)
    z_ref[...] = z
    zp_ref[...] = dinv * z


def _tc_mid(msg1, h, dinv, b1p, w2p):
    return pl.pallas_call(
        _mid_body,
        grid=(NP // BLK,),
        in_specs=[pl.BlockSpec((NC, BLK, DP), lambda i: (0, i, 0)),
                  pl.BlockSpec((BLK, DP), lambda i: (i, 0)),
                  pl.BlockSpec((BLK, 1), lambda i: (i, 0)),
                  pl.BlockSpec((1, DP), lambda i: (0, 0)),
                  pl.BlockSpec((DP, 1), lambda i: (0, 0))],
        out_specs=[pl.BlockSpec((BLK, 1), lambda i: (i, 0)),
                   pl.BlockSpec((BLK, 1), lambda i: (i, 0))],
        out_shape=[jax.ShapeDtypeStruct((NP, 1), f32),
                   jax.ShapeDtypeStruct((NP, 1), f32)],
    )(msg1, h, dinv, b1p, w2p)


def _fin_body(msg_ref, z_ref, dinv_ref, b2_ref, o_ref):
    dinv = dinv_ref[...]
    s = _colsum(msg_ref[...])
    o_ref[...] = dinv * s + (dinv * dinv) * z_ref[...] + b2_ref[...]


def _tc_fin(msg2, z, dinv, b2r):
    return pl.pallas_call(
        _fin_body,
        grid=(NP // BLK,),
        in_specs=[pl.BlockSpec((NW, BLK), lambda i: (0, i)),
                  pl.BlockSpec((BLK, 1), lambda i: (i, 0)),
                  pl.BlockSpec((BLK, 1), lambda i: (i, 0)),
                  pl.BlockSpec((1, 1), lambda i: (0, 0))],
        out_specs=pl.BlockSpec((BLK, 1), lambda i: (i, 0)),
        out_shape=jax.ShapeDtypeStruct((NP, 1), f32),
    )(msg2, z, dinv, b2r)


def _sc_mesh():
    return plsc.VectorSubcoreMesh(core_axis_name="c", subcore_axis_name="s")


@functools.cache
def _make_sc_scalar_msg():
    @functools.partial(
        pl.kernel,
        out_type=jax.ShapeDtypeStruct((NW, NP), f32),
        mesh=_sc_mesh(),
        compiler_params=pltpu.CompilerParams(needs_layout_passes=False),
        scratch_types=[
            pltpu.VMEM((NP,), f32),
            pltpu.VMEM((NP,), f32),
            pltpu.VMEM((NBLK, LB), i32),
            pltpu.VMEM((NBLK, LB), i32),
        ],
    )
    def k(vals_hbm, sidx_hbm, didx_hbm, out_hbm, vals, acc, sidx_all, didx_all):
        c = lax.axis_index("c")
        s = lax.axis_index("s")
        wid = c * NS + s
        zero = jnp.zeros((16,), f32)

        pltpu.sync_copy(vals_hbm, vals)
        pltpu.sync_copy(sidx_hbm.at[c, s], sidx_all)
        pltpu.sync_copy(didx_hbm.at[c, s], didx_all)

        @pl.loop(0, NP, step=16)
        def _(i):
            acc[pl.ds(i, 16)] = zero

        @pl.loop(0, NBLK)
        def _(j):
            @pl.loop(0, LB, step=16)
            def _(kk):
                sv = sidx_all[j, pl.ds(kk, 16)]
                dv = didx_all[j, pl.ds(kk, 16)]
                v = plsc.load_gather(vals, [sv])

                def cond(rem):
                    return jnp.any(rem)

                def body(rem):
                    _, last = plsc.scan_count(dv, mask=rem)
                    plsc.addupdate_scatter(acc, [dv], v, mask=last)
                    return rem & ~last

                lax.while_loop(cond, body,
                               jnp.full((16,), True, jnp.bool_))

        pltpu.sync_copy(acc, out_hbm.at[wid])

    return k


@functools.cache
def _make_sc_row_msg():
    @functools.partial(
        pl.kernel,
        out_type=jax.ShapeDtypeStruct((NC, NP, DP), f32),
        mesh=_sc_mesh(),
        scratch_types=[
            pltpu.VMEM((LB, DP), f32),
            pltpu.VMEM((LB, DP), f32),
            pltpu.VMEM((NBLK * LB,), i32),
            pltpu.VMEM((LB,), i32),
            pltpu.VMEM((LB,), i32),
            pltpu.VMEM_SHARED((NP, DP), f32),
            pltpu.SemaphoreType.DMA,
            pltpu.SemaphoreType.DMA,
            pltpu.SemaphoreType.DMA,
            pltpu.SemaphoreType.DMA,
        ],
    )
    def k(v_hbm, sidx_hbm, didx_hbm, out_hbm, rows_a, rows_b, sidx_all,
          didx_a, didx_b, acc, sem_a, sem_b, sem_da, sem_db):
        c = lax.axis_index("c")
        s = lax.axis_index("s")
        zero = jnp.zeros((16,), f32)

        @pl.loop(0, LB)
        def _(r):
            @pl.loop(0, DP, step=16)
            def _(cc):
                rows_a[r, pl.ds(cc, 16)] = zero

        for i in range(NZ):
            pltpu.sync_copy(rows_a, acc.at[pl.ds(s * RB + i * LB, LB)])
        plsc.subcore_barrier()

        pltpu.sync_copy(sidx_hbm.at[c, s], sidx_all)

        def start(j, rows, didx, sg, sd):
            pltpu.async_copy(v_hbm.at[sidx_all.at[pl.ds(j * LB, LB)]],
                             rows, sg)
            pltpu.async_copy(didx_hbm.at[c, s, j], didx, sd)

        def wait(rows, didx, sg, sd):
            pltpu.make_async_copy(v_hbm.at[sidx_all.at[pl.ds(0, LB)]],
                                  rows, sg).wait()
            pltpu.make_async_copy(didx_hbm.at[c, s, 0], didx, sd).wait()

        start(0, rows_a, didx_a, sem_a, sem_da)

        @pl.loop(0, NBLK - 2, step=2)
        def _(j):
            start(j + 1, rows_b, didx_b, sem_b, sem_db)
            wait(rows_a, didx_a, sem_a, sem_da)
            pltpu.sync_copy(rows_a, acc.at[didx_a], add=True)
            start(j + 2, rows_a, didx_a, sem_a, sem_da)
            wait(rows_b, didx_b, sem_b, sem_db)
            pltpu.sync_copy(rows_b, acc.at[didx_b], add=True)

        wait(rows_a, didx_a, sem_a, sem_da)
        pltpu.sync_copy(rows_a, acc.at[didx_a], add=True)

        plsc.subcore_barrier()
        for i in range(NZ):
            sl = pl.ds(s * RB + i * LB, LB)
            pltpu.sync_copy(acc.at[sl], out_hbm.at[c, sl])

    return k


def kernel(x, edge_index, W1, b1, W2, b2):
    ei = edge_index.astype(i32)
    pad = jnp.full((EP - E,), PAD, i32)
    src = jnp.concatenate([ei[0], pad]).reshape(NC, NS, NBLK, LB)
    dst = jnp.concatenate([ei[1], pad]).reshape(NC, NS, NBLK, LB)

    xp = jnp.pad(x, ((0, NP - N), (0, 0)))
    W1p = jnp.pad(W1, ((0, 0), (0, DP - DH)))
    b1p = jnp.pad(b1, (0, DP - DH)).reshape(1, DP)
    W2p = jnp.pad(W2, ((0, DP - DH), (0, 0)))
    b2r = b2.reshape(1, 1)
    ones_n = jnp.ones((NP,), f32)

    degp = _make_sc_scalar_msg()(ones_n, dst, dst)
    h = _tc_matmul(xp, W1p)
    hp, dinv = _tc_prep(degp, h)
    msg1 = _make_sc_row_msg()(hp, src.reshape(NC, NS, NBLK * LB), dst)
    zp, z = _tc_mid(msg1, h, dinv, b1p, W2p)
    msg2 = _make_sc_scalar_msg()(zp.reshape(NP), src, dst)
    out = _tc_fin(msg2, z, dinv, b2r)
    return out[:N]


# speedup vs baseline: 18.6702x; 1.1070x over previous
import functools

import jax
import jax.numpy as jnp
from jax import lax
from jax.experimental import pallas as pl
from jax.experimental.pallas import tpu as pltpu
from jax.experimental.pallas import tpu_sc as plsc

N = 10000
NP = 10240
DIN = 128
DH = 100
DP = 128
E = 320000
NC, NS, LB = 2, 16, 128
NW = NC * NS
F0, F1 = 121, 37
MAXF = F0
NBT = NS * (F0 + F1)
EP = NBT * LB
EPX = (NBT + MAXF) * LB
PAD = N
RB = NP // NS
NZ = RB // LB

f32 = jnp.float32
i32 = jnp.int32
BLK = 1280


def _mm_body(x_ref, w_ref, o_ref):
    o_ref[...] = jnp.dot(x_ref[...], w_ref[...], preferred_element_type=f32)


def _tc_matmul(xp, w):
    return pl.pallas_call(
        _mm_body,
        grid=(NP // BLK,),
        in_specs=[pl.BlockSpec((BLK, DIN), lambda i: (i, 0)),
                  pl.BlockSpec((DIN, DP), lambda i: (0, 0))],
        out_specs=pl.BlockSpec((BLK, DP), lambda i: (i, 0)),
        out_shape=jax.ShapeDtypeStruct((NP, DP), f32),
    )(xp, w)


def _colsum(part):
    ones = jnp.ones((NW, 1), f32)
    return lax.dot_general(part, ones, (((0,), (0,)), ((), ())),
                           preferred_element_type=f32)


def _prep_body(degp_ref, h_ref, hp_ref, dinv_ref):
    deg = _colsum(degp_ref[...]) + 1.0
    dinv = 1.0 / jnp.sqrt(deg)
    dinv_ref[...] = dinv
    hp_ref[...] = h_ref[...] * dinv


def _tc_prep(degp, h):
    return pl.pallas_call(
        _prep_body,
        grid=(NP // BLK,),
        in_specs=[pl.BlockSpec((NW, BLK), lambda i: (0, i)),
                  pl.BlockSpec((BLK, DP), lambda i: (i, 0))],
        out_specs=[pl.BlockSpec((BLK, DP), lambda i: (i, 0)),
                   pl.BlockSpec((BLK, 1), lambda i: (i, 0))],
        out_shape=[jax.ShapeDtypeStruct((NP, DP), f32),
                   jax.ShapeDtypeStruct((NP, 1), f32)],
    )(degp, h)


def _mid_body(msg_ref, h_ref, dinv_ref, b1_ref, w2_ref, zp_ref, z_ref):
    dinv = dinv_ref[...]
    ssum = msg_ref[0] + msg_ref[1]
    h1 = jnp.maximum(dinv * ssum + (dinv * dinv) * h_ref[...] + b1_ref[...],
                     0.0)
    z = jnp.dot(h1, w2_ref[...], preferred_element_type=f32)
    z_ref[...] = z
    zp_ref[...] = dinv * z


def _tc_mid(msg1, h, dinv, b1p, w2p):
    return pl.pallas_call(
        _mid_body,
        grid=(NP // BLK,),
        in_specs=[pl.BlockSpec((NC, BLK, DP), lambda i: (0, i, 0)),
                  pl.BlockSpec((BLK, DP), lambda i: (i, 0)),
                  pl.BlockSpec((BLK, 1), lambda i: (i, 0)),
                  pl.BlockSpec((1, DP), lambda i: (0, 0)),
                  pl.BlockSpec((DP, 1), lambda i: (0, 0))],
        out_specs=[pl.BlockSpec((BLK, 1), lambda i: (i, 0)),
                   pl.BlockSpec((BLK, 1), lambda i: (i, 0))],
        out_shape=[jax.ShapeDtypeStruct((NP, 1), f32),
                   jax.ShapeDtypeStruct((NP, 1), f32)],
    )(msg1, h, dinv, b1p, w2p)


def _fin_body(msg_ref, z_ref, dinv_ref, b2_ref, o_ref):
    dinv = dinv_ref[...]
    s = _colsum(msg_ref[...])
    o_ref[...] = dinv * s + (dinv * dinv) * z_ref[...] + b2_ref[...]


def _tc_fin(msg2, z, dinv, b2r):
    return pl.pallas_call(
        _fin_body,
        grid=(NP // BLK,),
        in_specs=[pl.BlockSpec((NW, BLK), lambda i: (0, i)),
                  pl.BlockSpec((BLK, 1), lambda i: (i, 0)),
                  pl.BlockSpec((BLK, 1), lambda i: (i, 0)),
                  pl.BlockSpec((1, 1), lambda i: (0, 0))],
        out_specs=pl.BlockSpec((BLK, 1), lambda i: (i, 0)),
        out_shape=jax.ShapeDtypeStruct((NP, 1), f32),
    )(msg2, z, dinv, b2r)


def _sc_mesh():
    return plsc.VectorSubcoreMesh(core_axis_name="c", subcore_axis_name="s")


@functools.cache
def _make_sc_scalar_msg():
    @functools.partial(
        pl.kernel,
        out_type=jax.ShapeDtypeStruct((NW, NP), f32),
        mesh=_sc_mesh(),
        compiler_params=pltpu.CompilerParams(needs_layout_passes=False),
        scratch_types=[
            pltpu.VMEM((NP,), f32),
            pltpu.VMEM((NP,), f32),
            pltpu.VMEM((MAXF * LB,), i32),
            pltpu.VMEM((MAXF * LB,), i32),
        ],
    )
    def k(vals_hbm, sidx_hbm, didx_hbm, out_hbm, vals, acc, sidx_all, didx_all):
        c = lax.axis_index("c")
        s = lax.axis_index("s")
        wid = c * NS + s
        base = jnp.where(c == 0, s * F0, NS * F0 + s * F1)
        cnt = jnp.where(c == 0, F0, F1)
        zero = jnp.zeros((16,), f32)

        pltpu.sync_copy(vals_hbm, vals)
        pltpu.sync_copy(sidx_hbm.at[pl.ds(base * LB, MAXF * LB)], sidx_all)
        pltpu.sync_copy(didx_hbm.at[pl.ds(base * LB, MAXF * LB)], didx_all)

        @pl.loop(0, NP, step=16)
        def _(i):
            acc[pl.ds(i, 16)] = zero

        @pl.loop(0, cnt * LB, step=16)
        def _(kk):
                sv = sidx_all[pl.ds(kk, 16)]
                dv = didx_all[pl.ds(kk, 16)]
                v = plsc.load_gather(vals, [sv])

                def cond(rem):
                    return jnp.any(rem)

                def body(rem):
                    _, last = plsc.scan_count(dv, mask=rem)
                    plsc.addupdate_scatter(acc, [dv], v, mask=last)
                    return rem & ~last

                lax.while_loop(cond, body,
                               jnp.full((16,), True, jnp.bool_))

        pltpu.sync_copy(acc, out_hbm.at[wid])

    return k


@functools.cache
def _make_sc_row_msg():
    @functools.partial(
        pl.kernel,
        out_type=jax.ShapeDtypeStruct((NC, NP, DP), f32),
        mesh=_sc_mesh(),
        scratch_types=[
            pltpu.VMEM((LB, DP), f32),
            pltpu.VMEM((LB, DP), f32),
            pltpu.VMEM((MAXF * LB,), i32),
            pltpu.VMEM((LB,), i32),
            pltpu.VMEM((LB,), i32),
            pltpu.VMEM_SHARED((NP, DP), f32),
            pltpu.SemaphoreType.DMA,
            pltpu.SemaphoreType.DMA,
            pltpu.SemaphoreType.DMA,
            pltpu.SemaphoreType.DMA,
        ],
    )
    def k(v_hbm, sidx_hbm, didx_hbm, out_hbm, rows_a, rows_b, sidx_all,
          didx_a, didx_b, acc, sem_a, sem_b, sem_da, sem_db):
        c = lax.axis_index("c")
        s = lax.axis_index("s")
        base = jnp.where(c == 0, s * F0, NS * F0 + s * F1)
        cnt = jnp.where(c == 0, F0, F1)
        zero = jnp.zeros((16,), f32)

        @pl.loop(0, LB)
        def _(r):
            @pl.loop(0, DP, step=16)
            def _(cc):
                rows_a[r, pl.ds(cc, 16)] = zero

        for i in range(NZ):
            pltpu.sync_copy(rows_a, acc.at[pl.ds(s * RB + i * LB, LB)])
        plsc.subcore_barrier()

        pltpu.sync_copy(sidx_hbm.at[pl.ds(base * LB, MAXF * LB)], sidx_all)

        def start(j, rows, didx, sg, sd):
            pltpu.async_copy(v_hbm.at[sidx_all.at[pl.ds(j * LB, LB)]],
                             rows, sg)
            pltpu.async_copy(didx_hbm.at[pl.ds((base + j) * LB, LB)], didx, sd)

        def wait(rows, didx, sg, sd):
            pltpu.make_async_copy(v_hbm.at[sidx_all.at[pl.ds(0, LB)]],
                                  rows, sg).wait()
            pltpu.make_async_copy(didx_hbm.at[pl.ds(0, LB)], didx, sd).wait()

        start(0, rows_a, didx_a, sem_a, sem_da)

        @pl.loop(0, cnt - 2, step=2)
        def _(j):
            start(j + 1, rows_b, didx_b, sem_b, sem_db)
            wait(rows_a, didx_a, sem_a, sem_da)
            pltpu.sync_copy(rows_a, acc.at[didx_a], add=True)
            start(j + 2, rows_a, didx_a, sem_a, sem_da)
            wait(rows_b, didx_b, sem_b, sem_db)
            pltpu.sync_copy(rows_b, acc.at[didx_b], add=True)

        wait(rows_a, didx_a, sem_a, sem_da)
        pltpu.sync_copy(rows_a, acc.at[didx_a], add=True)

        plsc.subcore_barrier()
        for i in range(NZ):
            sl = pl.ds(s * RB + i * LB, LB)
            pltpu.sync_copy(acc.at[sl], out_hbm.at[c, sl])

    return k


def kernel(x, edge_index, W1, b1, W2, b2):
    ei = edge_index.astype(i32)
    pad = jnp.full((EPX - E,), PAD, i32)
    srcf = jnp.concatenate([ei[0], pad])
    dstf = jnp.concatenate([ei[1], pad])

    xp = jnp.pad(x, ((0, NP - N), (0, 0)))
    W1p = jnp.pad(W1, ((0, 0), (0, DP - DH)))
    b1p = jnp.pad(b1, (0, DP - DH)).reshape(1, DP)
    W2p = jnp.pad(W2, ((0, DP - DH), (0, 0)))
    b2r = b2.reshape(1, 1)
    ones_n = jnp.ones((NP,), f32)

    degp = _make_sc_scalar_msg()(ones_n, dstf, dstf)
    h = _tc_matmul(xp, W1p)
    hp, dinv = _tc_prep(degp, h)
    msg1 = _make_sc_row_msg()(hp, srcf, dstf)
    zp, z = _tc_mid(msg1, h, dinv, b1p, W2p)
    msg2 = _make_sc_scalar_msg()(zp.reshape(NP), srcf, dstf)
    out = _tc_fin(msg2, z, dinv, b2r)
    return out[:N]


# speedup vs baseline: 20.3377x; 1.0893x over previous
import functools

import jax
import jax.numpy as jnp
from jax import lax
from jax.experimental import pallas as pl
from jax.experimental.pallas import tpu as pltpu
from jax.experimental.pallas import tpu_sc as plsc

N = 10000
NP = 10240
DIN = 128
DH = 100
DP = 128
E = 320000
NS = 16

LBS = 128
NBS = -(-E // (NS * LBS))
EPS = NS * NBS * LBS

LBR = 80
NBR = 251
EPR = NS * NBR * LBR

PAD = N
RB = NP // NS
NZ = RB // LBR

f32 = jnp.float32
i32 = jnp.int32
BLK = 1280


def _mm_body(x_ref, w_ref, o_ref):
    o_ref[...] = jnp.dot(x_ref[...], w_ref[...], preferred_element_type=f32)


def _tc_matmul(xp, w):
    return pl.pallas_call(
        _mm_body,
        grid=(NP // BLK,),
        in_specs=[pl.BlockSpec((BLK, DIN), lambda i: (i, 0)),
                  pl.BlockSpec((DIN, DP), lambda i: (0, 0))],
        out_specs=pl.BlockSpec((BLK, DP), lambda i: (i, 0)),
        out_shape=jax.ShapeDtypeStruct((NP, DP), f32),
    )(xp, w)


def _colsum(part):
    ones = jnp.ones((NS, 1), f32)
    return lax.dot_general(part, ones, (((0,), (0,)), ((), ())),
                           preferred_element_type=f32)


def _prep_body(degp_ref, h_ref, hp_ref, dinv_ref):
    deg = _colsum(degp_ref[...]) + 1.0
    dinv = 1.0 / jnp.sqrt(deg)
    dinv_ref[...] = dinv
    hp_ref[...] = h_ref[...] * dinv


def _tc_prep(degp, h):
    return pl.pallas_call(
        _prep_body,
        grid=(NP // BLK,),
        in_specs=[pl.BlockSpec((NS, BLK), lambda i: (0, i)),
                  pl.BlockSpec((BLK, DP), lambda i: (i, 0))],
        out_specs=[pl.BlockSpec((BLK, DP), lambda i: (i, 0)),
                   pl.BlockSpec((BLK, 1), lambda i: (i, 0))],
        out_shape=[jax.ShapeDtypeStruct((NP, DP), f32),
                   jax.ShapeDtypeStruct((NP, 1), f32)],
    )(degp, h)


def _mid_body(msg_ref, h_ref, dinv_ref, b1_ref, w2_ref, zp_ref, z_ref):
    dinv = dinv_ref[...]
    h1 = jnp.maximum(dinv * msg_ref[...] + (dinv * dinv) * h_ref[...]
                     + b1_ref[...], 0.0)
    z = jnp.dot(h1, w2_ref[...], preferred_element_type=f32)
    z_ref[...] = z
    zp_ref[...] = dinv * z


def _tc_mid(msg1, h, dinv, b1p, w2p):
    return pl.pallas_call(
        _mid_body,
        grid=(NP // BLK,),
        in_specs=[pl.BlockSpec((BLK, DP), lambda i: (i, 0)),
                  pl.BlockSpec((BLK, DP), lambda i: (i, 0)),
                  pl.BlockSpec((BLK, 1), lambda i: (i, 0)),
                  pl.BlockSpec((1, DP), lambda i: (0, 0)),
                  pl.BlockSpec((DP, 1), lambda i: (0, 0))],
        out_specs=[pl.BlockSpec((BLK, 1), lambda i: (i, 0)),
                   pl.BlockSpec((BLK, 1), lambda i: (i, 0))],
        out_shape=[jax.ShapeDtypeStruct((NP, 1), f32),
                   jax.ShapeDtypeStruct((NP, 1), f32)],
    )(msg1, h, dinv, b1p, w2p)


def _fin_body(msg_ref, z_ref, dinv_ref, b2_ref, o_ref):
    dinv = dinv_ref[...]
    s = _colsum(msg_ref[...])
    o_ref[...] = dinv * s + (dinv * dinv) * z_ref[...] + b2_ref[...]


def _tc_fin(msg2, z, dinv, b2r):
    return pl.pallas_call(
        _fin_body,
        grid=(NP // BLK,),
        in_specs=[pl.BlockSpec((NS, BLK), lambda i: (0, i)),
                  pl.BlockSpec((BLK, 1), lambda i: (i, 0)),
                  pl.BlockSpec((BLK, 1), lambda i: (i, 0)),
                  pl.BlockSpec((1, 1), lambda i: (0, 0))],
        out_specs=pl.BlockSpec((BLK, 1), lambda i: (i, 0)),
        out_shape=jax.ShapeDtypeStruct((NP, 1), f32),
    )(msg2, z, dinv, b2r)


def _sc_mesh():
    return plsc.VectorSubcoreMesh(core_axis_name="c", subcore_axis_name="s",
                                  num_cores=1)


@functools.cache
def _make_sc_scalar_msg():
    @functools.partial(
        pl.kernel,
        out_type=jax.ShapeDtypeStruct((NS, NP), f32),
        mesh=_sc_mesh(),
        compiler_params=pltpu.CompilerParams(needs_layout_passes=False),
        scratch_types=[
            pltpu.VMEM((NP,), f32),
            pltpu.VMEM((NP,), f32),
            pltpu.VMEM((NBS * LBS,), i32),
            pltpu.VMEM((NBS * LBS,), i32),
        ],
    )
    def k(vals_hbm, sidx_hbm, didx_hbm, out_hbm, vals, acc, sidx_all, didx_all):
        s = lax.axis_index("s")
        wpb = NBS * LBS
        zero = jnp.zeros((16,), f32)

        pltpu.sync_copy(vals_hbm, vals)
        pltpu.sync_copy(sidx_hbm.at[pl.ds(s * wpb, wpb)], sidx_all)
        pltpu.sync_copy(didx_hbm.at[pl.ds(s * wpb, wpb)], didx_all)

        @pl.loop(0, NP, step=16)
        def _(i):
            acc[pl.ds(i, 16)] = zero

        @pl.loop(0, wpb, step=16)
        def _(kk):
            sv = sidx_all[pl.ds(kk, 16)]
            dv = didx_all[pl.ds(kk, 16)]
            v = plsc.load_gather(vals, [sv])

            def cond(rem):
                return jnp.any(rem)

            def body(rem):
                _, last = plsc.scan_count(dv, mask=rem)
                plsc.addupdate_scatter(acc, [dv], v, mask=last)
                return rem & ~last

            lax.while_loop(cond, body, jnp.full((16,), True, jnp.bool_))

        pltpu.sync_copy(acc, out_hbm.at[s])

    return k


@functools.cache
def _make_sc_row_msg():
    @functools.partial(
        pl.kernel,
        out_type=jax.ShapeDtypeStruct((NP, DP), f32),
        mesh=_sc_mesh(),
        scratch_types=[
            pltpu.VMEM((LBR, DP), f32),
            pltpu.VMEM((LBR, DP), f32),
            pltpu.VMEM((NBR * LBR,), i32),
            pltpu.VMEM((LBR,), i32),
            pltpu.VMEM((LBR,), i32),
            pltpu.VMEM_SHARED((NP, DP), f32),
            pltpu.SemaphoreType.DMA,
            pltpu.SemaphoreType.DMA,
            pltpu.SemaphoreType.DMA,
            pltpu.SemaphoreType.DMA,
        ],
    )
    def k(v_hbm, sidx_hbm, didx_hbm, out_hbm, rows_a, rows_b, sidx_all,
          didx_a, didx_b, acc, sem_a, sem_b, sem_da, sem_db):
        s = lax.axis_index("s")
        wpb = NBR * LBR
        zero = jnp.zeros((16,), f32)

        @pl.loop(0, LBR)
        def _(r):
            @pl.loop(0, DP, step=16)
            def _(cc):
                rows_a[r, pl.ds(cc, 16)] = zero

        for i in range(NZ):
            pltpu.sync_copy(rows_a, acc.at[pl.ds(s * RB + i * LBR, LBR)])
        plsc.subcore_barrier()

        pltpu.sync_copy(sidx_hbm.at[pl.ds(s * wpb, wpb)], sidx_all)

        def start(j, rows, didx, sg, sd):
            pltpu.async_copy(v_hbm.at[sidx_all.at[pl.ds(j * LBR, LBR)]],
                             rows, sg)
            pltpu.async_copy(didx_hbm.at[pl.ds(s * wpb + j * LBR, LBR)],
                             didx, sd)

        def wait(rows, didx, sg, sd):
            pltpu.make_async_copy(v_hbm.at[sidx_all.at[pl.ds(0, LBR)]],
                                  rows, sg).wait()
            pltpu.make_async_copy(didx_hbm.at[pl.ds(0, LBR)], didx, sd).wait()

        start(0, rows_a, didx_a, sem_a, sem_da)

        @pl.loop(0, NBR - 2, step=2)
        def _(j):
            start(j + 1, rows_b, didx_b, sem_b, sem_db)
            wait(rows_a, didx_a, sem_a, sem_da)
            pltpu.sync_copy(rows_a, acc.at[didx_a], add=True)
            start(j + 2, rows_a, didx_a, sem_a, sem_da)
            wait(rows_b, didx_b, sem_b, sem_db)
            pltpu.sync_copy(rows_b, acc.at[didx_b], add=True)

        wait(rows_a, didx_a, sem_a, sem_da)
        pltpu.sync_copy(rows_a, acc.at[didx_a], add=True)

        plsc.subcore_barrier()
        for i in range(NZ):
            sl = pl.ds(s * RB + i * LBR, LBR)
            pltpu.sync_copy(acc.at[sl], out_hbm.at[sl])

    return k


def _padded(col, total):
    return jnp.concatenate([col, jnp.full((total - E,), PAD, i32)])


def kernel(x, edge_index, W1, b1, W2, b2):
    ei = edge_index.astype(i32)
    src_s = _padded(ei[0], EPS)
    dst_s = _padded(ei[1], EPS)
    src_r = _padded(ei[0], EPR)
    dst_r = _padded(ei[1], EPR)

    xp = jnp.pad(x, ((0, NP - N), (0, 0)))
    W1p = jnp.pad(W1, ((0, 0), (0, DP - DH)))
    b1p = jnp.pad(b1, (0, DP - DH)).reshape(1, DP)
    W2p = jnp.pad(W2, ((0, DP - DH), (0, 0)))
    b2r = b2.reshape(1, 1)
    ones_n = jnp.ones((NP,), f32)

    degp = _make_sc_scalar_msg()(ones_n, dst_s, dst_s)
    h = _tc_matmul(xp, W1p)
    hp, dinv = _tc_prep(degp, h)
    msg1 = _make_sc_row_msg()(hp, src_r, dst_r)
    zp, z = _tc_mid(msg1, h, dinv, b1p, W2p)
    msg2 = _make_sc_scalar_msg()(zp.reshape(NP), src_s, dst_s)
    out = _tc_fin(msg2, z, dinv, b2r)
    return out[:N]


# speedup vs baseline: 22.0180x; 1.0826x over previous
import functools

import jax
import jax.numpy as jnp
from jax import lax
from jax.experimental import pallas as pl
from jax.experimental.pallas import tpu as pltpu
from jax.experimental.pallas import tpu_sc as plsc

N = 10000
NP = 10240
DIN = 128
DH = 100
DP = 128
E = 320000
NS = 16
EPW = E // NS

LBR = 80
NBR = EPW // LBR

RB = NP // NS
NZ = RB // LBR

f32 = jnp.float32
i32 = jnp.int32
BLK = 1280


def _mm_body(x_ref, w_ref, o_ref):
    o_ref[...] = jnp.dot(x_ref[...], w_ref[...], preferred_element_type=f32)


def _tc_matmul(xp, w):
    return pl.pallas_call(
        _mm_body,
        grid=(NP // BLK,),
        in_specs=[pl.BlockSpec((BLK, DIN), lambda i: (i, 0)),
                  pl.BlockSpec((DIN, DP), lambda i: (0, 0))],
        out_specs=pl.BlockSpec((BLK, DP), lambda i: (i, 0)),
        out_shape=jax.ShapeDtypeStruct((NP, DP), f32),
    )(xp, w)


def _colsum(part):
    ones = jnp.ones((NS, 1), f32)
    return lax.dot_general(part, ones, (((0,), (0,)), ((), ())),
                           preferred_element_type=f32)


def _prep_body(degp_ref, h_ref, hp_ref, dinv_ref):
    deg = _colsum(degp_ref[...]) + 1.0
    dinv = 1.0 / jnp.sqrt(deg)
    dinv_ref[...] = dinv
    hp_ref[...] = h_ref[...] * dinv


def _tc_prep(degp, h):
    return pl.pallas_call(
        _prep_body,
        grid=(NP // BLK,),
        in_specs=[pl.BlockSpec((NS, BLK), lambda i: (0, i)),
                  pl.BlockSpec((BLK, DP), lambda i: (i, 0))],
        out_specs=[pl.BlockSpec((BLK, DP), lambda i: (i, 0)),
                   pl.BlockSpec((BLK, 1), lambda i: (i, 0))],
        out_shape=[jax.ShapeDtypeStruct((NP, DP), f32),
                   jax.ShapeDtypeStruct((NP, 1), f32)],
    )(degp, h)


def _mid_body(msg_ref, h_ref, dinv_ref, b1_ref, w2_ref, zp_ref, z_ref):
    dinv = dinv_ref[...]
    h1 = jnp.maximum(dinv * msg_ref[...] + (dinv * dinv) * h_ref[...]
                     + b1_ref[...], 0.0)
    z = jnp.dot(h1, w2_ref[...], preferred_element_type=f32)
    z_ref[...] = z
    zp_ref[...] = dinv * z


def _tc_mid(msg1, h, dinv, b1p, w2p):
    return pl.pallas_call(
        _mid_body,
        grid=(NP // BLK,),
        in_specs=[pl.BlockSpec((BLK, DP), lambda i: (i, 0)),
                  pl.BlockSpec((BLK, DP), lambda i: (i, 0)),
                  pl.BlockSpec((BLK, 1), lambda i: (i, 0)),
                  pl.BlockSpec((1, DP), lambda i: (0, 0)),
                  pl.BlockSpec((DP, 1), lambda i: (0, 0))],
        out_specs=[pl.BlockSpec((BLK, 1), lambda i: (i, 0)),
                   pl.BlockSpec((BLK, 1), lambda i: (i, 0))],
        out_shape=[jax.ShapeDtypeStruct((NP, 1), f32),
                   jax.ShapeDtypeStruct((NP, 1), f32)],
    )(msg1, h, dinv, b1p, w2p)


def _fin_body(msg_ref, z_ref, dinv_ref, b2_ref, o_ref):
    dinv = dinv_ref[...]
    s = _colsum(msg_ref[...])
    o_ref[...] = dinv * s + (dinv * dinv) * z_ref[...] + b2_ref[...]


def _tc_fin(msg2, z, dinv, b2r):
    return pl.pallas_call(
        _fin_body,
        grid=(NP // BLK,),
        in_specs=[pl.BlockSpec((NS, BLK), lambda i: (0, i)),
                  pl.BlockSpec((BLK, 1), lambda i: (i, 0)),
                  pl.BlockSpec((BLK, 1), lambda i: (i, 0)),
                  pl.BlockSpec((1, 1), lambda i: (0, 0))],
        out_specs=pl.BlockSpec((BLK, 1), lambda i: (i, 0)),
        out_shape=jax.ShapeDtypeStruct((NP, 1), f32),
    )(msg2, z, dinv, b2r)


def _sc_mesh():
    return plsc.VectorSubcoreMesh(core_axis_name="c", subcore_axis_name="s",
                                  num_cores=1)


@functools.cache
def _make_sc_scalar_msg():
    @functools.partial(
        pl.kernel,
        out_type=jax.ShapeDtypeStruct((NS, NP), f32),
        mesh=_sc_mesh(),
        compiler_params=pltpu.CompilerParams(needs_layout_passes=False),
        scratch_types=[
            pltpu.VMEM((NP,), f32),
            pltpu.VMEM((NP,), f32),
            pltpu.VMEM((EPW,), i32),
            pltpu.VMEM((EPW,), i32),
        ],
    )
    def k(vals_hbm, sidx_hbm, didx_hbm, out_hbm, vals, acc, sidx_all, didx_all):
        s = lax.axis_index("s")
        zero = jnp.zeros((16,), f32)

        pltpu.sync_copy(vals_hbm, vals)
        pltpu.sync_copy(sidx_hbm.at[pl.ds(s * EPW, EPW)], sidx_all)
        pltpu.sync_copy(didx_hbm.at[pl.ds(s * EPW, EPW)], didx_all)

        @pl.loop(0, NP, step=16)
        def _(i):
            acc[pl.ds(i, 16)] = zero

        @pl.loop(0, EPW, step=16)
        def _(kk):
            sv = sidx_all[pl.ds(kk, 16)]
            dv = didx_all[pl.ds(kk, 16)]
            v = plsc.load_gather(vals, [sv])

            def cond(rem):
                return jnp.any(rem)

            def body(rem):
                _, last = plsc.scan_count(dv, mask=rem)
                plsc.addupdate_scatter(acc, [dv], v, mask=last)
                return rem & ~last

            lax.while_loop(cond, body, jnp.full((16,), True, jnp.bool_))

        pltpu.sync_copy(acc, out_hbm.at[s])

    return k


@functools.cache
def _make_sc_row_msg():
    @functools.partial(
        pl.kernel,
        out_type=jax.ShapeDtypeStruct((NP, DP), f32),
        mesh=_sc_mesh(),
        scratch_types=[
            pltpu.VMEM((LBR, DP), f32),
            pltpu.VMEM((LBR, DP), f32),
            pltpu.VMEM((EPW,), i32),
            pltpu.VMEM((LBR,), i32),
            pltpu.VMEM((LBR,), i32),
            pltpu.VMEM_SHARED((NP, DP), f32),
            pltpu.SemaphoreType.DMA,
            pltpu.SemaphoreType.DMA,
            pltpu.SemaphoreType.DMA,
            pltpu.SemaphoreType.DMA,
            pltpu.SemaphoreType.DMA,
            pltpu.SemaphoreType.DMA,
        ],
    )
    def k(v_hbm, sidx_hbm, didx_hbm, out_hbm, rows_a, rows_b, sidx_all,
          didx_a, didx_b, acc, sem_ga, sem_gb, sem_sa, sem_sb, sem_da,
          sem_db):
        s = lax.axis_index("s")
        zero = jnp.zeros((16,), f32)

        @pl.loop(0, LBR)
        def _(r):
            @pl.loop(0, DP, step=16)
            def _(cc):
                rows_a[r, pl.ds(cc, 16)] = zero

        for i in range(NZ):
            pltpu.sync_copy(rows_a, acc.at[pl.ds(s * RB + i * LBR, LBR)])
        plsc.subcore_barrier()

        pltpu.sync_copy(sidx_hbm.at[pl.ds(s * EPW, EPW)], sidx_all)

        def start_g(j, rows, didx, sg, sd):
            pltpu.async_copy(v_hbm.at[sidx_all.at[pl.ds(j * LBR, LBR)]],
                             rows, sg)
            pltpu.async_copy(didx_hbm.at[pl.ds(s * EPW + j * LBR, LBR)],
                             didx, sd)

        def wait_g(rows, didx, sg, sd):
            pltpu.make_async_copy(v_hbm.at[sidx_all.at[pl.ds(0, LBR)]],
                                  rows, sg).wait()
            pltpu.make_async_copy(didx_hbm.at[pl.ds(0, LBR)], didx, sd).wait()

        def start_s(rows, didx, ss):
            pltpu.async_copy(rows, acc.at[didx], ss, add=True)

        def wait_s(rows, didx, ss):
            pltpu.make_async_copy(rows, acc.at[didx], ss).wait()

        start_g(0, rows_a, didx_a, sem_ga, sem_da)
        start_g(1, rows_b, didx_b, sem_gb, sem_db)

        @pl.loop(0, NBR, step=2)
        def _(j):
            wait_g(rows_a, didx_a, sem_ga, sem_da)
            start_s(rows_a, didx_a, sem_sa)
            wait_g(rows_b, didx_b, sem_gb, sem_db)
            start_s(rows_b, didx_b, sem_sb)
            wait_s(rows_a, didx_a, sem_sa)

            @pl.when(j + 2 < NBR)
            def _():
                start_g(j + 2, rows_a, didx_a, sem_ga, sem_da)

            wait_s(rows_b, didx_b, sem_sb)

            @pl.when(j + 3 < NBR)
            def _():
                start_g(j + 3, rows_b, didx_b, sem_gb, sem_db)

        plsc.subcore_barrier()
        for i in range(NZ):
            sl = pl.ds(s * RB + i * LBR, LBR)
            pltpu.sync_copy(acc.at[sl], out_hbm.at[sl])

    return k


def kernel(x, edge_index, W1, b1, W2, b2):
    ei = edge_index.astype(i32)
    src = ei[0]
    dst = ei[1]

    xp = jnp.pad(x, ((0, NP - N), (0, 0)))
    W1p = jnp.pad(W1, ((0, 0), (0, DP - DH)))
    b1p = jnp.pad(b1, (0, DP - DH)).reshape(1, DP)
    W2p = jnp.pad(W2, ((0, DP - DH), (0, 0)))
    b2r = b2.reshape(1, 1)
    ones_n = jnp.ones((NP,), f32)

    degp = _make_sc_scalar_msg()(ones_n, dst, dst)
    h = _tc_matmul(xp, W1p)
    hp, dinv = _tc_prep(degp, h)
    msg1 = _make_sc_row_msg()(hp, src, dst)
    zp, z = _tc_mid(msg1, h, dinv, b1p, W2p)
    msg2 = _make_sc_scalar_msg()(zp.reshape(NP), src, dst)
    out = _tc_fin(msg2, z, dinv, b2r)
    return out[:N]


# speedup vs baseline: 24.9384x; 1.1326x over previous
import functools

import jax
import jax.numpy as jnp
from jax import lax
from jax.experimental import pallas as pl
from jax.experimental.pallas import tpu as pltpu
from jax.experimental.pallas import tpu_sc as plsc

N = 10000
NP = 10240
DIN = 128
DH = 100
DP = 128
E = 320000
NS = 16
EPW = E // NS

LBR = 104
NBR = 193
EPR = NBR * LBR
PAD = N

RB = NP // NS
NZF = RB // LBR
RBT = RB - NZF * LBR

f32 = jnp.float32
i32 = jnp.int32
BLK = 1280


def _mm_body(x_ref, w_ref, o_ref):
    o_ref[...] = jnp.dot(x_ref[...], w_ref[...], preferred_element_type=f32)


def _tc_matmul(xp, w):
    return pl.pallas_call(
        _mm_body,
        grid=(NP // BLK,),
        in_specs=[pl.BlockSpec((BLK, DIN), lambda i: (i, 0)),
                  pl.BlockSpec((DIN, DP), lambda i: (0, 0))],
        out_specs=pl.BlockSpec((BLK, DP), lambda i: (i, 0)),
        out_shape=jax.ShapeDtypeStruct((NP, DP), f32),
    )(xp, w)


def _colsum(part):
    ones = jnp.ones((NS, 1), f32)
    return lax.dot_general(part, ones, (((0,), (0,)), ((), ())),
                           preferred_element_type=f32)


def _prep_body(degp_ref, h_ref, hp_ref, dinv_ref):
    deg = _colsum(degp_ref[...]) + 1.0
    dinv = 1.0 / jnp.sqrt(deg)
    dinv_ref[...] = dinv
    hp_ref[...] = h_ref[...] * dinv


def _tc_prep(degp, h):
    return pl.pallas_call(
        _prep_body,
        grid=(NP // BLK,),
        in_specs=[pl.BlockSpec((NS, BLK), lambda i: (0, i)),
                  pl.BlockSpec((BLK, DP), lambda i: (i, 0))],
        out_specs=[pl.BlockSpec((BLK, DP), lambda i: (i, 0)),
                   pl.BlockSpec((BLK, 1), lambda i: (i, 0))],
        out_shape=[jax.ShapeDtypeStruct((NP, DP), f32),
                   jax.ShapeDtypeStruct((NP, 1), f32)],
    )(degp, h)


def _mid_body(msg_ref, h_ref, dinv_ref, b1_ref, w2_ref, zp_ref, z_ref):
    dinv = dinv_ref[...]
    h1 = jnp.maximum(dinv * msg_ref[...] + (dinv * dinv) * h_ref[...]
                     + b1_ref[...], 0.0)
    z = jnp.dot(h1, w2_ref[...], preferred_element_type=f32)
    z_ref[...] = z
    zp_ref[...] = dinv * z


def _tc_mid(msg1, h, dinv, b1p, w2p):
    return pl.pallas_call(
        _mid_body,
        grid=(NP // BLK,),
        in_specs=[pl.BlockSpec((BLK, DP), lambda i: (i, 0)),
                  pl.BlockSpec((BLK, DP), lambda i: (i, 0)),
                  pl.BlockSpec((BLK, 1), lambda i: (i, 0)),
                  pl.BlockSpec((1, DP), lambda i: (0, 0)),
                  pl.BlockSpec((DP, 1), lambda i: (0, 0))],
        out_specs=[pl.BlockSpec((BLK, 1), lambda i: (i, 0)),
                   pl.BlockSpec((BLK, 1), lambda i: (i, 0))],
        out_shape=[jax.ShapeDtypeStruct((NP, 1), f32),
                   jax.ShapeDtypeStruct((NP, 1), f32)],
    )(msg1, h, dinv, b1p, w2p)


def _fin_body(msg_ref, z_ref, dinv_ref, b2_ref, o_ref):
    dinv = dinv_ref[...]
    s = _colsum(msg_ref[...])
    o_ref[...] = dinv * s + (dinv * dinv) * z_ref[...] + b2_ref[...]


def _tc_fin(msg2, z, dinv, b2r):
    return pl.pallas_call(
        _fin_body,
        grid=(NP // BLK,),
        in_specs=[pl.BlockSpec((NS, BLK), lambda i: (0, i)),
                  pl.BlockSpec((BLK, 1), lambda i: (i, 0)),
                  pl.BlockSpec((BLK, 1), lambda i: (i, 0)),
                  pl.BlockSpec((1, 1), lambda i: (0, 0))],
        out_specs=pl.BlockSpec((BLK, 1), lambda i: (i, 0)),
        out_shape=jax.ShapeDtypeStruct((NP, 1), f32),
    )(msg2, z, dinv, b2r)


def _sc_mesh():
    return plsc.VectorSubcoreMesh(core_axis_name="c", subcore_axis_name="s",
                                  num_cores=1)


@functools.cache
def _make_sc_scalar_msg():
    @functools.partial(
        pl.kernel,
        out_type=jax.ShapeDtypeStruct((NS, NP), f32),
        mesh=_sc_mesh(),
        compiler_params=pltpu.CompilerParams(needs_layout_passes=False),
        scratch_types=[
            pltpu.VMEM((NP,), f32),
            pltpu.VMEM((NP,), f32),
            pltpu.VMEM((EPW,), i32),
            pltpu.VMEM((EPW,), i32),
        ],
    )
    def k(vals_hbm, sidx_hbm, didx_hbm, out_hbm, vals, acc, sidx_all, didx_all):
        s = lax.axis_index("s")
        zero = jnp.zeros((16,), f32)

        pltpu.sync_copy(vals_hbm, vals)
        pltpu.sync_copy(sidx_hbm.at[pl.ds(s * EPW, EPW)], sidx_all)
        pltpu.sync_copy(didx_hbm.at[pl.ds(s * EPW, EPW)], didx_all)

        @pl.loop(0, NP, step=16)
        def _(i):
            acc[pl.ds(i, 16)] = zero

        @pl.loop(0, EPW, step=16)
        def _(kk):
            sv = sidx_all[pl.ds(kk, 16)]
            dv = didx_all[pl.ds(kk, 16)]
            v = plsc.load_gather(vals, [sv])

            def cond(rem):
                return jnp.any(rem)

            def body(rem):
                _, last = plsc.scan_count(dv, mask=rem)
                plsc.addupdate_scatter(acc, [dv], v, mask=last)
                return rem & ~last

            lax.while_loop(cond, body, jnp.full((16,), True, jnp.bool_))

        pltpu.sync_copy(acc, out_hbm.at[s])

    return k


@functools.cache
def _make_sc_row_msg():
    @functools.partial(
        pl.kernel,
        out_type=jax.ShapeDtypeStruct((NP, DP), f32),
        mesh=_sc_mesh(),
        scratch_types=[
            pltpu.VMEM((LBR, DP), f32),
            pltpu.VMEM((LBR, DP), f32),
            pltpu.VMEM((EPR,), i32),
            pltpu.VMEM((LBR,), i32),
            pltpu.VMEM((LBR,), i32),
            pltpu.VMEM_SHARED((NP, DP), f32),
            pltpu.SemaphoreType.DMA,
            pltpu.SemaphoreType.DMA,
            pltpu.SemaphoreType.DMA,
            pltpu.SemaphoreType.DMA,
        ],
    )
    def k(v_hbm, sidx_hbm, didx_hbm, out_hbm, rows_a, rows_b, sidx_all,
          didx_a, didx_b, acc, sem_a, sem_b, sem_da, sem_db):
        s = lax.axis_index("s")
        zero = jnp.zeros((16,), f32)

        @pl.loop(0, LBR)
        def _(r):
            @pl.loop(0, DP, step=16)
            def _(cc):
                rows_a[r, pl.ds(cc, 16)] = zero

        for i in range(NZF):
            pltpu.sync_copy(rows_a, acc.at[pl.ds(s * RB + i * LBR, LBR)])
        pltpu.sync_copy(rows_a.at[pl.ds(0, RBT)],
                        acc.at[pl.ds(s * RB + NZF * LBR, RBT)])
        plsc.subcore_barrier()

        pltpu.sync_copy(sidx_hbm.at[pl.ds(s * EPR, EPR)], sidx_all)

        def start(j, rows, didx, sg, sd):
            pltpu.async_copy(v_hbm.at[sidx_all.at[pl.ds(j * LBR, LBR)]],
                             rows, sg)
            pltpu.async_copy(didx_hbm.at[pl.ds(s * EPR + j * LBR, LBR)],
                             didx, sd)

        def wait(rows, didx, sg, sd):
            pltpu.make_async_copy(v_hbm.at[sidx_all.at[pl.ds(0, LBR)]],
                                  rows, sg).wait()
            pltpu.make_async_copy(didx_hbm.at[pl.ds(0, LBR)], didx, sd).wait()

        start(0, rows_a, didx_a, sem_a, sem_da)

        @pl.loop(0, NBR - 2, step=2)
        def _(j):
            start(j + 1, rows_b, didx_b, sem_b, sem_db)
            wait(rows_a, didx_a, sem_a, sem_da)
            pltpu.sync_copy(rows_a, acc.at[didx_a], add=True)
            start(j + 2, rows_a, didx_a, sem_a, sem_da)
            wait(rows_b, didx_b, sem_b, sem_db)
            pltpu.sync_copy(rows_b, acc.at[didx_b], add=True)

        wait(rows_a, didx_a, sem_a, sem_da)
        pltpu.sync_copy(rows_a, acc.at[didx_a], add=True)

        plsc.subcore_barrier()
        for i in range(NZF):
            sl = pl.ds(s * RB + i * LBR, LBR)
            pltpu.sync_copy(acc.at[sl], out_hbm.at[sl])
        slt = pl.ds(s * RB + NZF * LBR, RBT)
        pltpu.sync_copy(acc.at[slt], out_hbm.at[slt])

    return k


def kernel(x, edge_index, W1, b1, W2, b2):
    ei = edge_index.astype(i32)
    src = ei[0]
    dst = ei[1]
    src_r = jnp.pad(src.reshape(NS, EPW), ((0, 0), (0, EPR - EPW)),
                    constant_values=PAD).reshape(-1)
    dst_r = jnp.pad(dst.reshape(NS, EPW), ((0, 0), (0, EPR - EPW)),
                    constant_values=PAD).reshape(-1)

    xp = jnp.pad(x, ((0, NP - N), (0, 0)))
    W1p = jnp.pad(W1, ((0, 0), (0, DP - DH)))
    b1p = jnp.pad(b1, (0, DP - DH)).reshape(1, DP)
    W2p = jnp.pad(W2, ((0, DP - DH), (0, 0)))
    b2r = b2.reshape(1, 1)
    ones_n = jnp.ones((NP,), f32)

    degp = _make_sc_scalar_msg()(ones_n, dst, dst)
    h = _tc_matmul(xp, W1p)
    hp, dinv = _tc_prep(degp, h)
    msg1 = _make_sc_row_msg()(hp, src_r, dst_r)
    zp, z = _tc_mid(msg1, h, dinv, b1p, W2p)
    msg2 = _make_sc_scalar_msg()(zp.reshape(NP), src, dst)
    out = _tc_fin(msg2, z, dinv, b2r)
    return out[:N]


# speedup vs baseline: 29.1513x; 1.1689x over previous
import functools

import jax
import jax.numpy as jnp
from jax import lax
from jax.experimental import pallas as pl
from jax.experimental.pallas import tpu as pltpu
from jax.experimental.pallas import tpu_sc as plsc

N = 10000
NP = 10240
DIN = 128
DH = 100
DP = 128
E = 320000
NS = 16
EPW = E // NS

LBR = 104
NBR = 193
EPR = NBR * LBR
PAD = N

RB = NP // NS
NZF = RB // LBR
RBT = RB - NZF * LBR

f32 = jnp.float32
i32 = jnp.int32
BLK = 1280


def _mm_body(x_ref, w_ref, o_ref):
    o_ref[...] = jnp.dot(x_ref[...], w_ref[...], preferred_element_type=f32)


def _tc_matmul(xp, w):
    return pl.pallas_call(
        _mm_body,
        grid=(NP // BLK,),
        in_specs=[pl.BlockSpec((BLK, DIN), lambda i: (i, 0)),
                  pl.BlockSpec((DIN, DP), lambda i: (0, 0))],
        out_specs=pl.BlockSpec((BLK, DP), lambda i: (i, 0)),
        out_shape=jax.ShapeDtypeStruct((NP, DP), f32),
    )(xp, w)


def _colsum(part):
    ones = jnp.ones((NS, 1), f32)
    return lax.dot_general(part, ones, (((0,), (0,)), ((), ())),
                           preferred_element_type=f32)


def _prep_body(degp_ref, h_ref, hp_ref, dinv_ref):
    deg = _colsum(degp_ref[...]) + 1.0
    dinv = 1.0 / jnp.sqrt(deg)
    dinv_ref[...] = dinv
    hp_ref[...] = h_ref[...] * dinv


def _tc_prep(degp, h):
    return pl.pallas_call(
        _prep_body,
        grid=(NP // BLK,),
        in_specs=[pl.BlockSpec((NS, BLK), lambda i: (0, i)),
                  pl.BlockSpec((BLK, DP), lambda i: (i, 0))],
        out_specs=[pl.BlockSpec((BLK, DP), lambda i: (i, 0)),
                   pl.BlockSpec((BLK, 1), lambda i: (i, 0))],
        out_shape=[jax.ShapeDtypeStruct((NP, DP), f32),
                   jax.ShapeDtypeStruct((NP, 1), f32)],
    )(degp, h)


def _mid_body(msg_ref, h_ref, dinv_ref, b1_ref, w2_ref, zp_ref, z_ref):
    dinv = dinv_ref[...]
    h1 = jnp.maximum(dinv * msg_ref[...] + (dinv * dinv) * h_ref[...]
                     + b1_ref[...], 0.0)
    z = jnp.dot(h1, w2_ref[...], preferred_element_type=f32)
    z_ref[...] = z
    zp_ref[...] = dinv * z


def _tc_mid(msg1, h, dinv, b1p, w2p):
    return pl.pallas_call(
        _mid_body,
        grid=(NP // BLK,),
        in_specs=[pl.BlockSpec((BLK, DP), lambda i: (i, 0)),
                  pl.BlockSpec((BLK, DP), lambda i: (i, 0)),
                  pl.BlockSpec((BLK, 1), lambda i: (i, 0)),
                  pl.BlockSpec((1, DP), lambda i: (0, 0)),
                  pl.BlockSpec((DP, 1), lambda i: (0, 0))],
        out_specs=[pl.BlockSpec((BLK, 1), lambda i: (i, 0)),
                   pl.BlockSpec((BLK, 1), lambda i: (i, 0))],
        out_shape=[jax.ShapeDtypeStruct((NP, 1), f32),
                   jax.ShapeDtypeStruct((NP, 1), f32)],
    )(msg1, h, dinv, b1p, w2p)


def _fin_body(msg_ref, z_ref, dinv_ref, b2_ref, o_ref):
    dinv = dinv_ref[...]
    s = _colsum(msg_ref[...])
    o_ref[...] = dinv * s + (dinv * dinv) * z_ref[...] + b2_ref[...]


def _tc_fin(msg2, z, dinv, b2r):
    return pl.pallas_call(
        _fin_body,
        grid=(NP // BLK,),
        in_specs=[pl.BlockSpec((NS, BLK), lambda i: (0, i)),
                  pl.BlockSpec((BLK, 1), lambda i: (i, 0)),
                  pl.BlockSpec((BLK, 1), lambda i: (i, 0)),
                  pl.BlockSpec((1, 1), lambda i: (0, 0))],
        out_specs=pl.BlockSpec((BLK, 1), lambda i: (i, 0)),
        out_shape=jax.ShapeDtypeStruct((NP, 1), f32),
    )(msg2, z, dinv, b2r)


def _sc_mesh():
    return plsc.VectorSubcoreMesh(core_axis_name="c", subcore_axis_name="s",
                                  num_cores=1)


@functools.cache
def _make_sc_scalar_msg():
    @functools.partial(
        pl.kernel,
        out_type=jax.ShapeDtypeStruct((NS, NP), f32),
        mesh=_sc_mesh(),
        compiler_params=pltpu.CompilerParams(needs_layout_passes=False),
        scratch_types=[
            pltpu.VMEM((NP,), f32),
            pltpu.VMEM((NP,), f32),
            pltpu.VMEM((EPW,), i32),
            pltpu.VMEM((EPW,), i32),
        ],
    )
    def k(vals_hbm, sidx_hbm, didx_hbm, out_hbm, vals, acc, sidx_all, didx_all):
        s = lax.axis_index("s")
        zero = jnp.zeros((16,), f32)

        pltpu.sync_copy(vals_hbm, vals)
        pltpu.sync_copy(sidx_hbm.at[pl.ds(s * EPW, EPW)], sidx_all)
        pltpu.sync_copy(didx_hbm.at[pl.ds(s * EPW, EPW)], didx_all)

        @pl.loop(0, NP, step=16)
        def _(i):
            acc[pl.ds(i, 16)] = zero

        @pl.loop(0, EPW, step=32)
        def _(kk):
            sv1 = sidx_all[pl.ds(kk, 16)]
            dv1 = didx_all[pl.ds(kk, 16)]
            sv2 = sidx_all[pl.ds(kk + 16, 16)]
            dv2 = didx_all[pl.ds(kk + 16, 16)]
            v1 = plsc.load_gather(vals, [sv1])
            v2 = plsc.load_gather(vals, [sv2])
            _, l1 = plsc.scan_count(dv1)
            _, l2 = plsc.scan_count(dv2)
            plsc.addupdate_scatter(acc, [dv1], v1, mask=l1)
            plsc.addupdate_scatter(acc, [dv2], v2, mask=l2)
            r1 = ~l1
            r2 = ~l2

            @pl.when(jnp.any(r1 | r2))
            def _():
                def cond(carry):
                    c1, c2 = carry
                    return jnp.any(c1 | c2)

                def body(carry):
                    c1, c2 = carry
                    _, m1 = plsc.scan_count(dv1, mask=c1)
                    _, m2 = plsc.scan_count(dv2, mask=c2)
                    plsc.addupdate_scatter(acc, [dv1], v1, mask=m1 & c1)
                    plsc.addupdate_scatter(acc, [dv2], v2, mask=m2 & c2)
                    return (c1 & ~m1, c2 & ~m2)

                lax.while_loop(cond, body, (r1, r2))

        pltpu.sync_copy(acc, out_hbm.at[s])

    return k


@functools.cache
def _make_sc_row_msg():
    @functools.partial(
        pl.kernel,
        out_type=jax.ShapeDtypeStruct((NP, DP), f32),
        mesh=_sc_mesh(),
        scratch_types=[
            pltpu.VMEM((LBR, DP), f32),
            pltpu.VMEM((LBR, DP), f32),
            pltpu.VMEM((EPR,), i32),
            pltpu.VMEM((LBR,), i32),
            pltpu.VMEM((LBR,), i32),
            pltpu.VMEM_SHARED((NP, DP), f32),
            pltpu.SemaphoreType.DMA,
            pltpu.SemaphoreType.DMA,
            pltpu.SemaphoreType.DMA,
            pltpu.SemaphoreType.DMA,
        ],
    )
    def k(v_hbm, sidx_hbm, didx_hbm, out_hbm, rows_a, rows_b, sidx_all,
          didx_a, didx_b, acc, sem_a, sem_b, sem_da, sem_db):
        s = lax.axis_index("s")
        zero = jnp.zeros((16,), f32)

        @pl.loop(0, LBR)
        def _(r):
            @pl.loop(0, DP, step=16)
            def _(cc):
                rows_a[r, pl.ds(cc, 16)] = zero

        for i in range(NZF):
            pltpu.sync_copy(rows_a, acc.at[pl.ds(s * RB + i * LBR, LBR)])
        pltpu.sync_copy(rows_a.at[pl.ds(0, RBT)],
                        acc.at[pl.ds(s * RB + NZF * LBR, RBT)])
        plsc.subcore_barrier()

        pltpu.sync_copy(sidx_hbm.at[pl.ds(s * EPR, EPR)], sidx_all)

        def start(j, rows, didx, sg, sd):
            pltpu.async_copy(v_hbm.at[sidx_all.at[pl.ds(j * LBR, LBR)]],
                             rows, sg)
            pltpu.async_copy(didx_hbm.at[pl.ds(s * EPR + j * LBR, LBR)],
                             didx, sd)

        def wait(rows, didx, sg, sd):
            pltpu.make_async_copy(v_hbm.at[sidx_all.at[pl.ds(0, LBR)]],
                                  rows, sg).wait()
            pltpu.make_async_copy(didx_hbm.at[pl.ds(0, LBR)], didx, sd).wait()

        start(0, rows_a, didx_a, sem_a, sem_da)

        @pl.loop(0, NBR - 2, step=2)
        def _(j):
            start(j + 1, rows_b, didx_b, sem_b, sem_db)
            wait(rows_a, didx_a, sem_a, sem_da)
            pltpu.sync_copy(rows_a, acc.at[didx_a], add=True)
            start(j + 2, rows_a, didx_a, sem_a, sem_da)
            wait(rows_b, didx_b, sem_b, sem_db)
            pltpu.sync_copy(rows_b, acc.at[didx_b], add=True)

        wait(rows_a, didx_a, sem_a, sem_da)
        pltpu.sync_copy(rows_a, acc.at[didx_a], add=True)

        plsc.subcore_barrier()
        for i in range(NZF):
            sl = pl.ds(s * RB + i * LBR, LBR)
            pltpu.sync_copy(acc.at[sl], out_hbm.at[sl])
        slt = pl.ds(s * RB + NZF * LBR, RBT)
        pltpu.sync_copy(acc.at[slt], out_hbm.at[slt])

    return k


def kernel(x, edge_index, W1, b1, W2, b2):
    ei = edge_index.astype(i32)
    src = ei[0]
    dst = ei[1]
    src_r = jnp.pad(src.reshape(NS, EPW), ((0, 0), (0, EPR - EPW)),
                    constant_values=PAD).reshape(-1)
    dst_r = jnp.pad(dst.reshape(NS, EPW), ((0, 0), (0, EPR - EPW)),
                    constant_values=PAD).reshape(-1)

    xp = jnp.pad(x, ((0, NP - N), (0, 0)))
    W1p = jnp.pad(W1, ((0, 0), (0, DP - DH)))
    b1p = jnp.pad(b1, (0, DP - DH)).reshape(1, DP)
    W2p = jnp.pad(W2, ((0, DP - DH), (0, 0)))
    b2r = b2.reshape(1, 1)
    ones_n = jnp.ones((NP,), f32)

    degp = _make_sc_scalar_msg()(ones_n, dst, dst)
    h = _tc_matmul(xp, W1p)
    hp, dinv = _tc_prep(degp, h)
    msg1 = _make_sc_row_msg()(hp, src_r, dst_r)
    zp, z = _tc_mid(msg1, h, dinv, b1p, W2p)
    msg2 = _make_sc_scalar_msg()(zp.reshape(NP), src, dst)
    out = _tc_fin(msg2, z, dinv, b2r)
    return out[:N]


# speedup vs baseline: 31.2729x; 1.0728x over previous
import functools

import jax
import jax.numpy as jnp
from jax import lax
from jax.experimental import pallas as pl
from jax.experimental.pallas import tpu as pltpu
from jax.experimental.pallas import tpu_sc as plsc

N = 10000
NP = 10240
DIN = 128
DH = 100
DP = 128
E = 320000
NS = 16
EPW = E // NS

LBR = 104
NBR = 193
EPR = NBR * LBR
PAD = N

RB = NP // NS
NZF = RB // LBR
RBT = RB - NZF * LBR

f32 = jnp.float32
i32 = jnp.int32
BLK = 1280


def _mm_body(x_ref, w_ref, o_ref):
    o_ref[...] = jnp.dot(x_ref[...], w_ref[...], preferred_element_type=f32)


def _tc_matmul(xp, w):
    return pl.pallas_call(
        _mm_body,
        grid=(NP // BLK,),
        in_specs=[pl.BlockSpec((BLK, DIN), lambda i: (i, 0)),
                  pl.BlockSpec((DIN, DP), lambda i: (0, 0))],
        out_specs=pl.BlockSpec((BLK, DP), lambda i: (i, 0)),
        out_shape=jax.ShapeDtypeStruct((NP, DP), f32),
    )(xp, w)


def _colsum(part):
    ones = jnp.ones((NS, 1), f32)
    return lax.dot_general(part, ones, (((0,), (0,)), ((), ())),
                           preferred_element_type=f32)


def _prep_body(degp_ref, h_ref, hp_ref, dinv_ref):
    deg = _colsum(degp_ref[...]) + 1.0
    dinv = 1.0 / jnp.sqrt(deg)
    dinv_ref[...] = dinv
    hp_ref[...] = h_ref[...] * dinv


def _tc_prep(degp, h):
    return pl.pallas_call(
        _prep_body,
        grid=(NP // BLK,),
        in_specs=[pl.BlockSpec((NS, BLK), lambda i: (0, i)),
                  pl.BlockSpec((BLK, DP), lambda i: (i, 0))],
        out_specs=[pl.BlockSpec((BLK, DP), lambda i: (i, 0)),
                   pl.BlockSpec((BLK, 1), lambda i: (i, 0))],
        out_shape=[jax.ShapeDtypeStruct((NP, DP), f32),
                   jax.ShapeDtypeStruct((NP, 1), f32)],
    )(degp, h)


def _mid_body(msg_ref, h_ref, dinv_ref, b1_ref, w2_ref, zp_ref, z_ref):
    dinv = dinv_ref[...]
    h1 = jnp.maximum(dinv * msg_ref[...] + (dinv * dinv) * h_ref[...]
                     + b1_ref[...], 0.0)
    z = jnp.dot(h1, w2_ref[...], preferred_element_type=f32)
    z_ref[...] = z
    zp_ref[...] = dinv * z


def _tc_mid(msg1, h, dinv, b1p, w2p):
    return pl.pallas_call(
        _mid_body,
        grid=(NP // BLK,),
        in_specs=[pl.BlockSpec((BLK, DP), lambda i: (i, 0)),
                  pl.BlockSpec((BLK, DP), lambda i: (i, 0)),
                  pl.BlockSpec((BLK, 1), lambda i: (i, 0)),
                  pl.BlockSpec((1, DP), lambda i: (0, 0)),
                  pl.BlockSpec((DP, 1), lambda i: (0, 0))],
        out_specs=[pl.BlockSpec((BLK, 1), lambda i: (i, 0)),
                   pl.BlockSpec((BLK, 1), lambda i: (i, 0))],
        out_shape=[jax.ShapeDtypeStruct((NP, 1), f32),
                   jax.ShapeDtypeStruct((NP, 1), f32)],
    )(msg1, h, dinv, b1p, w2p)


def _fin_body(msg_ref, z_ref, dinv_ref, b2_ref, o_ref):
    dinv = dinv_ref[...]
    s = _colsum(msg_ref[...])
    o_ref[...] = dinv * s + (dinv * dinv) * z_ref[...] + b2_ref[...]


def _tc_fin(msg2, z, dinv, b2r):
    return pl.pallas_call(
        _fin_body,
        grid=(NP // BLK,),
        in_specs=[pl.BlockSpec((NS, BLK), lambda i: (0, i)),
                  pl.BlockSpec((BLK, 1), lambda i: (i, 0)),
                  pl.BlockSpec((BLK, 1), lambda i: (i, 0)),
                  pl.BlockSpec((1, 1), lambda i: (0, 0))],
        out_specs=pl.BlockSpec((BLK, 1), lambda i: (i, 0)),
        out_shape=jax.ShapeDtypeStruct((NP, 1), f32),
    )(msg2, z, dinv, b2r)


def _sc_mesh():
    return plsc.VectorSubcoreMesh(core_axis_name="c", subcore_axis_name="s",
                                  num_cores=1)


def _scalar_chunks(acc, didx_all, kk, vs, dvs):
    ls = [plsc.scan_count(dv)[1] for dv in dvs]
    for dv, v, l in zip(dvs, vs, ls):
        plsc.addupdate_scatter(acc, [dv], v, mask=l)
    rs = [~l for l in ls]
    anyrem = functools.reduce(lambda a, b: a | b, rs)

    @pl.when(jnp.any(anyrem))
    def _():
        def cond(carry):
            return jnp.any(functools.reduce(lambda a, b: a | b, carry))

        def body(carry):
            out = []
            for dv, v, c in zip(dvs, vs, carry):
                _, m = plsc.scan_count(dv, mask=c)
                plsc.addupdate_scatter(acc, [dv], v, mask=m & c)
                out.append(c & ~m)
            return tuple(out)

        lax.while_loop(cond, body, tuple(rs))


@functools.cache
def _make_sc_scalar_msg(use_ones):
    scratch = [
        pltpu.VMEM((NP,), f32),
        pltpu.VMEM((EPW,), i32),
    ]
    if not use_ones:
        scratch = [pltpu.VMEM((NP,), f32)] + scratch + \
                  [pltpu.VMEM((EPW,), i32)]

    @functools.partial(
        pl.kernel,
        out_type=jax.ShapeDtypeStruct((NS, NP), f32),
        mesh=_sc_mesh(),
        compiler_params=pltpu.CompilerParams(needs_layout_passes=False),
        scratch_types=scratch,
    )
    def k(*args):
        if use_ones:
            didx_hbm, out_hbm, acc, didx_all = args
        else:
            vals_hbm, sidx_hbm, didx_hbm, out_hbm, vals, acc, didx_all, \
                sidx_all = args
        s = lax.axis_index("s")
        zero = jnp.zeros((16,), f32)
        one = jnp.ones((16,), f32)

        if not use_ones:
            pltpu.sync_copy(vals_hbm, vals)
            pltpu.sync_copy(sidx_hbm.at[pl.ds(s * EPW, EPW)], sidx_all)
        pltpu.sync_copy(didx_hbm.at[pl.ds(s * EPW, EPW)], didx_all)

        @pl.loop(0, NP, step=16)
        def _(i):
            acc[pl.ds(i, 16)] = zero

        def group(kk, nch):
            dvs = [didx_all[pl.ds(kk + 16 * t, 16)] for t in range(nch)]
            if use_ones:
                vs = [one] * nch
            else:
                vs = [plsc.load_gather(vals, [sidx_all[pl.ds(kk + 16 * t, 16)]])
                      for t in range(nch)]
            _scalar_chunks(acc, didx_all, kk, vs, dvs)

        @pl.loop(0, EPW - 32, step=64)
        def _(kk):
            group(kk, 4)

        group(EPW - 32, 2)

        pltpu.sync_copy(acc, out_hbm.at[s])

    return k


@functools.cache
def _make_sc_row_msg():
    @functools.partial(
        pl.kernel,
        out_type=jax.ShapeDtypeStruct((NP, DP), f32),
        mesh=_sc_mesh(),
        scratch_types=[
            pltpu.VMEM((LBR, DP), f32),
            pltpu.VMEM((LBR, DP), f32),
            pltpu.VMEM((EPR,), i32),
            pltpu.VMEM((LBR,), i32),
            pltpu.VMEM((LBR,), i32),
            pltpu.VMEM_SHARED((NP, DP), f32),
            pltpu.SemaphoreType.DMA,
            pltpu.SemaphoreType.DMA,
            pltpu.SemaphoreType.DMA,
            pltpu.SemaphoreType.DMA,
        ],
    )
    def k(v_hbm, sidx_hbm, didx_hbm, out_hbm, rows_a, rows_b, sidx_all,
          didx_a, didx_b, acc, sem_a, sem_b, sem_da, sem_db):
        s = lax.axis_index("s")
        zero = jnp.zeros((16,), f32)

        @pl.loop(0, LBR)
        def _(r):
            @pl.loop(0, DP, step=16)
            def _(cc):
                rows_a[r, pl.ds(cc, 16)] = zero

        for i in range(NZF):
            pltpu.sync_copy(rows_a, acc.at[pl.ds(s * RB + i * LBR, LBR)])
        pltpu.sync_copy(rows_a.at[pl.ds(0, RBT)],
                        acc.at[pl.ds(s * RB + NZF * LBR, RBT)])
        plsc.subcore_barrier()

        pltpu.sync_copy(sidx_hbm.at[pl.ds(s * EPR, EPR)], sidx_all)

        def start(j, rows, didx, sg, sd):
            pltpu.async_copy(v_hbm.at[sidx_all.at[pl.ds(j * LBR, LBR)]],
                             rows, sg)
            pltpu.async_copy(didx_hbm.at[pl.ds(s * EPR + j * LBR, LBR)],
                             didx, sd)

        def wait(rows, didx, sg, sd):
            pltpu.make_async_copy(v_hbm.at[sidx_all.at[pl.ds(0, LBR)]],
                                  rows, sg).wait()
            pltpu.make_async_copy(didx_hbm.at[pl.ds(0, LBR)], didx, sd).wait()

        start(0, rows_a, didx_a, sem_a, sem_da)

        @pl.loop(0, NBR - 2, step=2)
        def _(j):
            start(j + 1, rows_b, didx_b, sem_b, sem_db)
            wait(rows_a, didx_a, sem_a, sem_da)
            pltpu.sync_copy(rows_a, acc.at[didx_a], add=True)
            start(j + 2, rows_a, didx_a, sem_a, sem_da)
            wait(rows_b, didx_b, sem_b, sem_db)
            pltpu.sync_copy(rows_b, acc.at[didx_b], add=True)

        wait(rows_a, didx_a, sem_a, sem_da)
        pltpu.sync_copy(rows_a, acc.at[didx_a], add=True)

        plsc.subcore_barrier()
        for i in range(NZF):
            sl = pl.ds(s * RB + i * LBR, LBR)
            pltpu.sync_copy(acc.at[sl], out_hbm.at[sl])
        slt = pl.ds(s * RB + NZF * LBR, RBT)
        pltpu.sync_copy(acc.at[slt], out_hbm.at[slt])

    return k


def kernel(x, edge_index, W1, b1, W2, b2):
    ei = edge_index.astype(i32)
    src = ei[0]
    dst = ei[1]
    src_r = jnp.pad(src.reshape(NS, EPW), ((0, 0), (0, EPR - EPW)),
                    constant_values=PAD).reshape(-1)
    dst_r = jnp.pad(dst.reshape(NS, EPW), ((0, 0), (0, EPR - EPW)),
                    constant_values=PAD).reshape(-1)

    xp = jnp.pad(x, ((0, NP - N), (0, 0)))
    W1p = jnp.pad(W1, ((0, 0), (0, DP - DH)))
    b1p = jnp.pad(b1, (0, DP - DH)).reshape(1, DP)
    W2p = jnp.pad(W2, ((0, DP - DH), (0, 0)))
    b2r = b2.reshape(1, 1)

    degp = _make_sc_scalar_msg(True)(dst)
    h = _tc_matmul(xp, W1p)
    hp, dinv = _tc_prep(degp, h)
    msg1 = _make_sc_row_msg()(hp, src_r, dst_r)
    zp, z = _tc_mid(msg1, h, dinv, b1p, W2p)
    msg2 = _make_sc_scalar_msg(False)(zp.reshape(NP), src, dst)
    out = _tc_fin(msg2, z, dinv, b2r)
    return out[:N]


# speedup vs baseline: 32.0530x; 1.0249x over previous
import functools

import jax
import jax.numpy as jnp
from jax import lax
from jax.experimental import pallas as pl
from jax.experimental.pallas import tpu as pltpu
from jax.experimental.pallas import tpu_sc as plsc

N = 10000
NP = 10240
DIN = 128
DH = 100
DP = 128
E = 320000
NS = 16
EPW = E // NS

LBR = 104
NBR = 193
EPR = NBR * LBR
PAD = N

RB = NP // NS
NZF = RB // LBR
RBT = RB - NZF * LBR

f32 = jnp.float32
i32 = jnp.int32
BLK = 2560


def _mm_body(x_ref, w_ref, o_ref):
    o_ref[...] = jnp.dot(x_ref[...], w_ref[...], preferred_element_type=f32)


def _tc_matmul(xp, w):
    return pl.pallas_call(
        _mm_body,
        grid=(NP // BLK,),
        in_specs=[pl.BlockSpec((BLK, DIN), lambda i: (i, 0)),
                  pl.BlockSpec((DIN, DP), lambda i: (0, 0))],
        out_specs=pl.BlockSpec((BLK, DP), lambda i: (i, 0)),
        out_shape=jax.ShapeDtypeStruct((NP, DP), f32),
    )(xp, w)


def _colsum(part):
    ones = jnp.ones((NS, 1), f32)
    return lax.dot_general(part, ones, (((0,), (0,)), ((), ())),
                           preferred_element_type=f32)


def _prep_body(degp_ref, h_ref, hp_ref, dinv_ref):
    deg = _colsum(degp_ref[...]) + 1.0
    dinv = 1.0 / jnp.sqrt(deg)
    dinv_ref[...] = dinv
    hp_ref[...] = h_ref[...] * dinv


def _tc_prep(degp, h):
    return pl.pallas_call(
        _prep_body,
        grid=(NP // BLK,),
        in_specs=[pl.BlockSpec((NS, BLK), lambda i: (0, i)),
                  pl.BlockSpec((BLK, DP), lambda i: (i, 0))],
        out_specs=[pl.BlockSpec((BLK, DP), lambda i: (i, 0)),
                   pl.BlockSpec((BLK, 1), lambda i: (i, 0))],
        out_shape=[jax.ShapeDtypeStruct((NP, DP), f32),
                   jax.ShapeDtypeStruct((NP, 1), f32)],
    )(degp, h)


def _mid_body(msg_ref, h_ref, dinv_ref, b1_ref, w2_ref, zp_ref, z_ref):
    dinv = dinv_ref[...]
    h1 = jnp.maximum(dinv * msg_ref[...] + (dinv * dinv) * h_ref[...]
                     + b1_ref[...], 0.0)
    z = jnp.dot(h1, w2_ref[...], preferred_element_type=f32)
    z_ref[...] = z
    zp_ref[...] = dinv * z


def _tc_mid(msg1, h, dinv, b1p, w2p):
    return pl.pallas_call(
        _mid_body,
        grid=(NP // BLK,),
        in_specs=[pl.BlockSpec((BLK, DP), lambda i: (i, 0)),
                  pl.BlockSpec((BLK, DP), lambda i: (i, 0)),
                  pl.BlockSpec((BLK, 1), lambda i: (i, 0)),
                  pl.BlockSpec((1, DP), lambda i: (0, 0)),
                  pl.BlockSpec((DP, 1), lambda i: (0, 0))],
        out_specs=[pl.BlockSpec((BLK, 1), lambda i: (i, 0)),
                   pl.BlockSpec((BLK, 1), lambda i: (i, 0))],
        out_shape=[jax.ShapeDtypeStruct((NP, 1), f32),
                   jax.ShapeDtypeStruct((NP, 1), f32)],
    )(msg1, h, dinv, b1p, w2p)


def _fin_body(msg_ref, z_ref, dinv_ref, b2_ref, o_ref):
    dinv = dinv_ref[...]
    s = _colsum(msg_ref[...])
    o_ref[...] = dinv * s + (dinv * dinv) * z_ref[...] + b2_ref[...]


def _tc_fin(msg2, z, dinv, b2r):
    return pl.pallas_call(
        _fin_body,
        grid=(NP // BLK,),
        in_specs=[pl.BlockSpec((NS, BLK), lambda i: (0, i)),
                  pl.BlockSpec((BLK, 1), lambda i: (i, 0)),
                  pl.BlockSpec((BLK, 1), lambda i: (i, 0)),
                  pl.BlockSpec((1, 1), lambda i: (0, 0))],
        out_specs=pl.BlockSpec((BLK, 1), lambda i: (i, 0)),
        out_shape=jax.ShapeDtypeStruct((NP, 1), f32),
    )(msg2, z, dinv, b2r)


def _sc_mesh():
    return plsc.VectorSubcoreMesh(core_axis_name="c", subcore_axis_name="s",
                                  num_cores=1)


def _scalar_chunks(acc, didx_all, kk, vs, dvs):
    ls = [plsc.scan_count(dv)[1] for dv in dvs]
    for dv, v, l in zip(dvs, vs, ls):
        plsc.addupdate_scatter(acc, [dv], v, mask=l)
    rs = [~l for l in ls]
    anyrem = functools.reduce(lambda a, b: a | b, rs)

    @pl.when(jnp.any(anyrem))
    def _():
        def cond(carry):
            return jnp.any(functools.reduce(lambda a, b: a | b, carry))

        def body(carry):
            out = []
            for dv, v, c in zip(dvs, vs, carry):
                _, m = plsc.scan_count(dv, mask=c)
                plsc.addupdate_scatter(acc, [dv], v, mask=m & c)
                out.append(c & ~m)
            return tuple(out)

        lax.while_loop(cond, body, tuple(rs))


@functools.cache
def _make_sc_scalar_msg(use_ones):
    scratch = [
        pltpu.VMEM((NP,), f32),
        pltpu.VMEM((EPW,), i32),
    ]
    if not use_ones:
        scratch = [pltpu.VMEM((NP,), f32)] + scratch + \
                  [pltpu.VMEM((EPW,), i32)]

    @functools.partial(
        pl.kernel,
        out_type=jax.ShapeDtypeStruct((NS, NP), f32),
        mesh=_sc_mesh(),
        compiler_params=pltpu.CompilerParams(needs_layout_passes=False),
        scratch_types=scratch,
    )
    def k(*args):
        if use_ones:
            didx_hbm, out_hbm, acc, didx_all = args
        else:
            vals_hbm, sidx_hbm, didx_hbm, out_hbm, vals, acc, didx_all, \
                sidx_all = args
        s = lax.axis_index("s")
        zero = jnp.zeros((16,), f32)
        one = jnp.ones((16,), f32)

        if not use_ones:
            pltpu.sync_copy(vals_hbm, vals)
            pltpu.sync_copy(sidx_hbm.at[pl.ds(s * EPW, EPW)], sidx_all)
        pltpu.sync_copy(didx_hbm.at[pl.ds(s * EPW, EPW)], didx_all)

        @pl.loop(0, NP, step=16)
        def _(i):
            acc[pl.ds(i, 16)] = zero

        def group(kk, nch):
            dvs = [didx_all[pl.ds(kk + 16 * t, 16)] for t in range(nch)]
            if use_ones:
                vs = [one] * nch
            else:
                vs = [plsc.load_gather(vals, [sidx_all[pl.ds(kk + 16 * t, 16)]])
                      for t in range(nch)]
            _scalar_chunks(acc, didx_all, kk, vs, dvs)

        @pl.loop(0, EPW - 32, step=64)
        def _(kk):
            group(kk, 4)

        group(EPW - 32, 2)

        pltpu.sync_copy(acc, out_hbm.at[s])

    return k


@functools.cache
def _make_sc_row_msg():
    @functools.partial(
        pl.kernel,
        out_type=jax.ShapeDtypeStruct((NP, DP), f32),
        mesh=_sc_mesh(),
        scratch_types=[
            pltpu.VMEM((LBR, DP), f32),
            pltpu.VMEM((LBR, DP), f32),
            pltpu.VMEM((EPR,), i32),
            pltpu.VMEM((LBR,), i32),
            pltpu.VMEM((LBR,), i32),
            pltpu.VMEM_SHARED((NP, DP), f32),
            pltpu.SemaphoreType.DMA,
            pltpu.SemaphoreType.DMA,
            pltpu.SemaphoreType.DMA,
            pltpu.SemaphoreType.DMA,
        ],
    )
    def k(v_hbm, sidx_hbm, didx_hbm, out_hbm, rows_a, rows_b, sidx_all,
          didx_a, didx_b, acc, sem_a, sem_b, sem_da, sem_db):
        s = lax.axis_index("s")
        zero = jnp.zeros((16,), f32)

        @pl.loop(0, LBR)
        def _(r):
            @pl.loop(0, DP, step=16)
            def _(cc):
                rows_a[r, pl.ds(cc, 16)] = zero

        for i in range(NZF):
            pltpu.sync_copy(rows_a, acc.at[pl.ds(s * RB + i * LBR, LBR)])
        pltpu.sync_copy(rows_a.at[pl.ds(0, RBT)],
                        acc.at[pl.ds(s * RB + NZF * LBR, RBT)])
        plsc.subcore_barrier()

        pltpu.sync_copy(sidx_hbm.at[pl.ds(s * EPR, EPR)], sidx_all)

        def start(j, rows, didx, sg, sd):
            pltpu.async_copy(v_hbm.at[sidx_all.at[pl.ds(j * LBR, LBR)]],
                             rows, sg)
            pltpu.async_copy(didx_hbm.at[pl.ds(s * EPR + j * LBR, LBR)],
                             didx, sd)

        def wait(rows, didx, sg, sd):
            pltpu.make_async_copy(v_hbm.at[sidx_all.at[pl.ds(0, LBR)]],
                                  rows, sg).wait()
            pltpu.make_async_copy(didx_hbm.at[pl.ds(0, LBR)], didx, sd).wait()

        start(0, rows_a, didx_a, sem_a, sem_da)

        @pl.loop(0, NBR - 2, step=2)
        def _(j):
            start(j + 1, rows_b, didx_b, sem_b, sem_db)
            wait(rows_a, didx_a, sem_a, sem_da)
            pltpu.sync_copy(rows_a, acc.at[didx_a], add=True)
            start(j + 2, rows_a, didx_a, sem_a, sem_da)
            wait(rows_b, didx_b, sem_b, sem_db)
            pltpu.sync_copy(rows_b, acc.at[didx_b], add=True)

        wait(rows_a, didx_a, sem_a, sem_da)
        pltpu.sync_copy(rows_a, acc.at[didx_a], add=True)

        plsc.subcore_barrier()
        for i in range(NZF):
            sl = pl.ds(s * RB + i * LBR, LBR)
            pltpu.sync_copy(acc.at[sl], out_hbm.at[sl])
        slt = pl.ds(s * RB + NZF * LBR, RBT)
        pltpu.sync_copy(acc.at[slt], out_hbm.at[slt])

    return k


def kernel(x, edge_index, W1, b1, W2, b2):
    ei = edge_index.astype(i32)
    src = ei[0]
    dst = ei[1]
    src_r = jnp.pad(src.reshape(NS, EPW), ((0, 0), (0, EPR - EPW)),
                    constant_values=PAD).reshape(-1)
    dst_r = jnp.pad(dst.reshape(NS, EPW), ((0, 0), (0, EPR - EPW)),
                    constant_values=PAD).reshape(-1)

    xp = jnp.pad(x, ((0, NP - N), (0, 0)))
    W1p = jnp.pad(W1, ((0, 0), (0, DP - DH)))
    b1p = jnp.pad(b1, (0, DP - DH)).reshape(1, DP)
    W2p = jnp.pad(W2, ((0, DP - DH), (0, 0)))
    b2r = b2.reshape(1, 1)

    degp = _make_sc_scalar_msg(True)(dst)
    h = _tc_matmul(xp, W1p)
    hp, dinv = _tc_prep(degp, h)
    msg1 = _make_sc_row_msg()(hp, src_r, dst_r)
    zp, z = _tc_mid(msg1, h, dinv, b1p, W2p)
    msg2 = _make_sc_scalar_msg(False)(zp.reshape(NP), src, dst)
    out = _tc_fin(msg2, z, dinv, b2r)
    return out[:N]
